# Initial kernel scaffold; baseline (speedup 1.0000x reference)
import jax
import jax.numpy as jnp
from jax.experimental import pallas as pl


def kernel(x, edge_index, W_g1, b_g1, W_g2, b_g2, W_x1, b_x1, W_y1, b_y1, a_src1, a_dst1, W_x2, b_x2, W_y2, b_y2, a_src2, a_dst2):
    raise NotImplementedError("write your pallas kernel here")


# speedup vs baseline: 12.8481x; 12.8481x over previous
import functools

import jax
import jax.numpy as jnp
from jax import lax
from jax.experimental import pallas as pl
from jax.experimental.pallas import tpu as pltpu
from jax.experimental.pallas import tpu_sc as plsc

N = 10000
NP = 10240
E = 320000

NC = 2
NS = 16
NW = NC * NS
EPT = E // NW
K = 80
NCHUNK = EPT // K
RPT = NP // NS

_mesh = plsc.VectorSubcoreMesh(
    core_axis_name="c", subcore_axis_name="s", num_cores=NC, num_subcores=NS)


def _wid(c, s):
    return c * NS + s


def _sc_deg(dst_h, zeros_h, out_h, dstv, ones_v, acc, sem):
    c = lax.axis_index("c")
    s = lax.axis_index("s")
    wid = _wid(c, s)
    pltpu.sync_copy(zeros_h.at[pl.ds(s * RPT, RPT)], acc.at[pl.ds(s * RPT, RPT)])

    def fill(i, carry):
        ones_v[i, :] = jnp.ones((16,), jnp.float32)
        return carry

    lax.fori_loop(0, K, fill, 0)
    plsc.subcore_barrier()

    def step(i, carry):
        base = wid * EPT + i * K
        pltpu.sync_copy(dst_h.at[pl.ds(base, K)], dstv)
        pltpu.sync_copy(ones_v, acc.at[dstv], add=True)
        return carry

    lax.fori_loop(0, NCHUNK, step, 0)
    plsc.subcore_barrier()
    pltpu.sync_copy(acc.at[pl.ds(s * RPT, RPT)], out_h.at[c, pl.ds(s * RPT, RPT)])


def _run_deg(dst, zeros16):
    f = pl.kernel(
        _sc_deg,
        out_type=jax.ShapeDtypeStruct((NC, NP, 16), jnp.float32),
        mesh=_mesh,
        compiler_params=pltpu.CompilerParams(use_tc_tiling_on_sc=False, needs_layout_passes=False),
        scratch_types=[
            pltpu.VMEM((K,), jnp.int32),
            pltpu.VMEM((K, 16), jnp.float32),
            pltpu.VMEM_SHARED((NP, 16), jnp.float32),
            pltpu.SemaphoreType.DMA,
        ],
    )
    return f(dst, zeros16)


def _sc_spmm(src_h, dst_h, tbl_h, zeros_h, out_h, srcv, dstv, rows, acc, sem):
    c = lax.axis_index("c")
    s = lax.axis_index("s")
    wid = _wid(c, s)
    pltpu.sync_copy(zeros_h.at[pl.ds(s * RPT, RPT)], acc.at[pl.ds(s * RPT, RPT)])
    plsc.subcore_barrier()

    def step(i, carry):
        base = wid * EPT + i * K
        pltpu.sync_copy(src_h.at[pl.ds(base, K)], srcv)
        pltpu.sync_copy(dst_h.at[pl.ds(base, K)], dstv)
        pltpu.async_copy(tbl_h.at[srcv], rows, sem).wait()
        pltpu.sync_copy(rows, acc.at[dstv], add=True)
        return carry

    lax.fori_loop(0, NCHUNK, step, 0)
    plsc.subcore_barrier()
    pltpu.sync_copy(acc.at[pl.ds(s * RPT, RPT)], out_h.at[c, pl.ds(s * RPT, RPT)])


def _run_spmm(src, dst, table, zeros_w, width):
    f = pl.kernel(
        _sc_spmm,
        out_type=jax.ShapeDtypeStruct((NC, NP, width), jnp.float32),
        mesh=_mesh,
        compiler_params=pltpu.CompilerParams(use_tc_tiling_on_sc=False, needs_layout_passes=False),
        scratch_types=[
            pltpu.VMEM((K,), jnp.int32),
            pltpu.VMEM((K,), jnp.int32),
            pltpu.VMEM((K, width), jnp.float32),
            pltpu.VMEM_SHARED((NP, width), jnp.float32),
            pltpu.SemaphoreType.DMA,
        ],
    )
    return f(src, dst, table, zeros_w)


def _sc_att_scalar(src_h, dst_h, sv_h, dv_h, zeros_h, ex_h, den_h,
                   srcv, dstv, s_tbl, d_tbl, exv, rows16, acc, sem):
    c = lax.axis_index("c")
    s = lax.axis_index("s")
    wid = _wid(c, s)
    pltpu.sync_copy(zeros_h.at[pl.ds(s * RPT, RPT)], acc.at[pl.ds(s * RPT, RPT)])
    pltpu.sync_copy(sv_h, s_tbl)
    pltpu.sync_copy(dv_h, d_tbl)
    plsc.subcore_barrier()

    def step(i, carry):
        base = wid * EPT + i * K
        pltpu.sync_copy(src_h.at[pl.ds(base, K)], srcv)
        pltpu.sync_copy(dst_h.at[pl.ds(base, K)], dstv)
        for j in range(K // 16):
            si = srcv[pl.ds(j * 16, 16)]
            di = dstv[pl.ds(j * 16, 16)]
            sval = plsc.load_gather(s_tbl, [si])
            dval = plsc.load_gather(d_tbl, [di])
            e = sval + dval
            e = jnp.maximum(e, 0.2 * e)
            exv[pl.ds(j * 16, 16)] = jnp.exp(e)
        for r in range(K):
            idx = jnp.full((16,), r, jnp.int32)
            rows16[r, :] = plsc.load_gather(exv, [idx])
        pltpu.sync_copy(exv, ex_h.at[pl.ds(base, K)])
        pltpu.sync_copy(rows16, acc.at[dstv], add=True)
        return carry

    lax.fori_loop(0, NCHUNK, step, 0)
    plsc.subcore_barrier()
    pltpu.sync_copy(acc.at[pl.ds(s * RPT, RPT)], den_h.at[c, pl.ds(s * RPT, RPT)])


def _run_att_scalar(src, dst, s_vec, d_vec, zeros16):
    f = pl.kernel(
        _sc_att_scalar,
        out_type=[
            jax.ShapeDtypeStruct((E,), jnp.float32),
            jax.ShapeDtypeStruct((NC, NP, 16), jnp.float32),
        ],
        mesh=_mesh,
        compiler_params=pltpu.CompilerParams(use_tc_tiling_on_sc=False, needs_layout_passes=False),
        scratch_types=[
            pltpu.VMEM((K,), jnp.int32),
            pltpu.VMEM((K,), jnp.int32),
            pltpu.VMEM((N,), jnp.float32),
            pltpu.VMEM((N,), jnp.float32),
            pltpu.VMEM((K,), jnp.float32),
            pltpu.VMEM((K, 16), jnp.float32),
            pltpu.VMEM_SHARED((NP, 16), jnp.float32),
            pltpu.SemaphoreType.DMA,
        ],
    )
    return f(src, dst, s_vec, d_vec, zeros16)


def _sc_att_wide(width, src_h, dst_h, w_h, tbl_h, zeros_h, out_h,
                 srcv, dstv, wv, rows, acc, sem):
    c = lax.axis_index("c")
    s = lax.axis_index("s")
    wid = _wid(c, s)
    pltpu.sync_copy(zeros_h.at[pl.ds(s * RPT, RPT)], acc.at[pl.ds(s * RPT, RPT)])
    plsc.subcore_barrier()

    def step(i, carry):
        base = wid * EPT + i * K
        pltpu.sync_copy(src_h.at[pl.ds(base, K)], srcv)
        pltpu.sync_copy(dst_h.at[pl.ds(base, K)], dstv)
        pltpu.sync_copy(w_h.at[pl.ds(base, K)], wv)
        pltpu.async_copy(tbl_h.at[srcv], rows, sem).wait()

        def scale(r, carry2):
            cvec = plsc.load_gather(wv, [jnp.full((16,), r, jnp.int32)])
            for g in range(width // 16):
                rows[r, pl.ds(g * 16, 16)] = rows[r, pl.ds(g * 16, 16)] * cvec
            return carry2

        lax.fori_loop(0, K, scale, 0)
        pltpu.sync_copy(rows, acc.at[dstv], add=True)
        return carry

    lax.fori_loop(0, NCHUNK, step, 0)
    plsc.subcore_barrier()
    pltpu.sync_copy(acc.at[pl.ds(s * RPT, RPT)], out_h.at[c, pl.ds(s * RPT, RPT)])


def _run_att_wide(src, dst, ex, table, zeros_w, width):
    f = pl.kernel(
        functools.partial(_sc_att_wide, width),
        out_type=jax.ShapeDtypeStruct((NC, NP, width), jnp.float32),
        mesh=_mesh,
        compiler_params=pltpu.CompilerParams(use_tc_tiling_on_sc=False, needs_layout_passes=False),
        scratch_types=[
            pltpu.VMEM((K,), jnp.int32),
            pltpu.VMEM((K,), jnp.int32),
            pltpu.VMEM((K,), jnp.float32),
            pltpu.VMEM((K, width), jnp.float32),
            pltpu.VMEM_SHARED((NP, width), jnp.float32),
            pltpu.SemaphoreType.DMA,
        ],
    )
    return f(src, dst, ex, table, zeros_w)


BN = 1000
GRID = N // BN


def _row_spec(width):
    return pl.BlockSpec((BN, width), lambda i: (i, 0))


def _full_spec(shape):
    nd = len(shape)
    return pl.BlockSpec(shape, lambda i: (0,) * nd)


def _rsq_from(deg0_ref, deg1_ref):
    deg = jnp.maximum(deg0_ref[:, 0:1] + deg1_ref[:, 0:1], 1.0)
    return lax.rsqrt(deg)


def _tc1(x_ref, wg1_ref, wx1_ref, bx1_ref, deg0_ref, deg1_ref,
         t1s_ref, hx1_ref):
    xb = x_ref[...]
    rsq = _rsq_from(deg0_ref, deg1_ref)
    t1s_ref[...] = jnp.dot(xb, wg1_ref[...],
                           preferred_element_type=jnp.float32) * rsq
    hx1_ref[...] = jnp.dot(xb, wx1_ref[...],
                           preferred_element_type=jnp.float32) + bx1_ref[...]


def _tc2(p0_ref, p1_ref, bg1_ref, wg2_ref, deg0_ref, deg1_ref, t2s_ref):
    rsq = _rsq_from(deg0_ref, deg1_ref)
    h1 = jnp.maximum((p0_ref[...] + p1_ref[...]) * rsq + bg1_ref[...], 0.0)
    t2s_ref[...] = jnp.dot(h1, wg2_ref[...],
                           preferred_element_type=jnp.float32) * rsq


def _tc3(q0_ref, q1_ref, bg2_ref, wy1_ref, by1_ref, a1_ref,
         deg0_ref, deg1_ref, z_ref, hy1s_ref, sd1_ref):
    rsq = _rsq_from(deg0_ref, deg1_ref)
    z = (q0_ref[...] + q1_ref[...]) * rsq + bg2_ref[...]
    z_ref[...] = z
    zs = z - jnp.max(z, axis=-1, keepdims=True)
    ez = jnp.exp(zs)
    preds = ez / jnp.sum(ez, axis=-1, keepdims=True)
    hy1 = jnp.dot(preds, wy1_ref[...],
                  preferred_element_type=jnp.float32) + by1_ref[...]
    sd1_ref[...] = jnp.dot(hy1, a1_ref[...], preferred_element_type=jnp.float32)
    hy1s_ref[...] = hy1 * rsq


def _tc4(r0_ref, r1_ref, d10_ref, d11_ref, u0_ref, u1_ref,
         wx2_ref, bx2_ref, wy2_ref, by2_ref, a2_ref, deg0_ref, deg1_ref,
         hx2_ref, sd2_ref):
    rsq = _rsq_from(deg0_ref, deg1_ref)
    invd = 1.0 / (d10_ref[:, 0:1] + d11_ref[:, 0:1] + 1e-16)
    x1 = jnp.maximum((r0_ref[...] + r1_ref[...]) * invd, 0.0)
    hx2_ref[...] = jnp.dot(x1, wx2_ref[...],
                           preferred_element_type=jnp.float32) + bx2_ref[...]
    y1 = jnp.maximum((u0_ref[...] + u1_ref[...]) * rsq, 0.0)
    hy2 = jnp.dot(y1, wy2_ref[...],
                  preferred_element_type=jnp.float32) + by2_ref[...]
    sd2_ref[...] = jnp.dot(hy2, a2_ref[...], preferred_element_type=jnp.float32)


def _tc5(z_ref, v0_ref, v1_ref, d20_ref, d21_ref, out_ref):
    invd = 1.0 / (d20_ref[:, 0:1] + d21_ref[:, 0:1] + 1e-16)
    out_ref[...] = (z_ref[...] + (v0_ref[...] + v1_ref[...]) * invd) * 0.5


def kernel(x, edge_index, W_g1, b_g1, W_g2, b_g2, W_x1, b_x1, W_y1, b_y1,
           a_src1, a_dst1, W_x2, b_x2, W_y2, b_y2, a_src2, a_dst2):
    f32 = jnp.float32
    src = edge_index[0]
    dst = edge_index[1]

    zeros16 = jnp.zeros((NP, 16), f32)
    zeros64 = jnp.zeros((NP, 64), f32)
    zeros128 = jnp.zeros((NP, 128), f32)

    degp = _run_deg(dst, zeros16)
    deg0, deg1 = degp[0, :N], degp[1, :N]

    t1s, hx1 = pl.pallas_call(
        _tc1,
        grid=(GRID,),
        in_specs=[
            _row_spec(128), _full_spec((128, 128)), _full_spec((128, 128)),
            _full_spec((1, 128)), _row_spec(16), _row_spec(16),
        ],
        out_specs=[_row_spec(128), _row_spec(128)],
        out_shape=[
            jax.ShapeDtypeStruct((N, 128), f32),
            jax.ShapeDtypeStruct((N, 128), f32),
        ],
    )(x, W_g1, W_x1, b_x1.reshape(1, 128), deg0, deg1)

    p = _run_spmm(src, dst, t1s, zeros128, 128)

    t2s = pl.pallas_call(
        _tc2,
        grid=(GRID,),
        in_specs=[
            _row_spec(128), _row_spec(128), _full_spec((1, 128)),
            _full_spec((128, 64)), _row_spec(16), _row_spec(16),
        ],
        out_specs=_row_spec(64),
        out_shape=jax.ShapeDtypeStruct((N, 64), f32),
    )(p[0, :N], p[1, :N], b_g1.reshape(1, 128), W_g2, deg0, deg1)

    q = _run_spmm(src, dst, t2s, zeros64, 64)

    a1 = jnp.concatenate(
        [a_src1.reshape(64, 1), a_dst1.reshape(64, 1),
         jnp.zeros((64, 6), f32)], axis=1)
    z, hy1s, sd1 = pl.pallas_call(
        _tc3,
        grid=(GRID,),
        in_specs=[
            _row_spec(64), _row_spec(64), _full_spec((1, 64)),
            _full_spec((64, 64)), _full_spec((1, 64)), _full_spec((64, 8)),
            _row_spec(16), _row_spec(16),
        ],
        out_specs=[_row_spec(64), _row_spec(64), _row_spec(8)],
        out_shape=[
            jax.ShapeDtypeStruct((N, 64), f32),
            jax.ShapeDtypeStruct((N, 64), f32),
            jax.ShapeDtypeStruct((N, 8), f32),
        ],
    )(q[0, :N], q[1, :N], b_g2.reshape(1, 64), W_y1, b_y1.reshape(1, 64), a1,
      deg0, deg1)

    s1 = sd1[:, 0] + 0.0
    d1 = sd1[:, 1] + 0.0

    ex1, den1 = _run_att_scalar(src, dst, s1, d1, zeros16)
    r = _run_att_wide(src, dst, ex1, hx1, zeros128, 128)
    u = _run_spmm(src, dst, hy1s, zeros64, 64)

    a2 = jnp.concatenate(
        [a_src2.reshape(64, 1), a_dst2.reshape(64, 1),
         jnp.zeros((64, 6), f32)], axis=1)
    hx2, sd2 = pl.pallas_call(
        _tc4,
        grid=(GRID,),
        in_specs=[
            _row_spec(128), _row_spec(128), _row_spec(16), _row_spec(16),
            _row_spec(64), _row_spec(64),
            _full_spec((128, 64)), _full_spec((1, 64)),
            _full_spec((64, 64)), _full_spec((1, 64)), _full_spec((64, 8)),
            _row_spec(16), _row_spec(16),
        ],
        out_specs=[_row_spec(64), _row_spec(8)],
        out_shape=[
            jax.ShapeDtypeStruct((N, 64), f32),
            jax.ShapeDtypeStruct((N, 8), f32),
        ],
    )(r[0, :N], r[1, :N], den1[0, :N], den1[1, :N], u[0, :N], u[1, :N],
      W_x2, b_x2.reshape(1, 64), W_y2, b_y2.reshape(1, 64), a2, deg0, deg1)

    s2 = sd2[:, 0] + 0.0
    d2 = sd2[:, 1] + 0.0

    ex2, den2 = _run_att_scalar(src, dst, s2, d2, zeros16)
    v = _run_att_wide(src, dst, ex2, hx2, zeros64, 64)

    out = pl.pallas_call(
        _tc5,
        grid=(GRID,),
        in_specs=[
            _row_spec(64), _row_spec(64), _row_spec(64),
            _row_spec(16), _row_spec(16),
        ],
        out_specs=_row_spec(64),
        out_shape=jax.ShapeDtypeStruct((N, 64), f32),
    )(z, v[0, :N], v[1, :N], den2[0, :N], den2[1, :N])
    return out


# speedup vs baseline: 27.5173x; 2.1417x over previous
import functools

import jax
import jax.numpy as jnp
from jax import lax
from jax.experimental import pallas as pl
from jax.experimental.pallas import tpu as pltpu
from jax.experimental.pallas import tpu_sc as plsc

N = 10000
NP = 10240
E = 320000

NC = 2
NS = 16
NW = NC * NS
EPT = E // NW
RPT = NP // NS

KD = 80
NCHD = EPT // KD
KS = 40
NCHS = EPT // KS
DEPTH = 6
KA = 16
NCHA = EPT // KA
NB = 5
NROUND = NCHA // NB

_mesh = plsc.VectorSubcoreMesh(
    core_axis_name="c", subcore_axis_name="s", num_cores=NC, num_subcores=NS)

_sc_params = pltpu.CompilerParams(
    use_tc_tiling_on_sc=False, needs_layout_passes=False)


def _zero_vmem_1d(ref, n):
    zv = jnp.zeros((16,), jnp.float32)

    def body(i, carry):
        ref[pl.ds(i * 16, 16)] = zv
        return carry

    lax.fori_loop(0, n // 16, body, 0)


def _sc_deg(dst2_h, out_h, dstv, deg_v, stage, tmp_v, out_v, sem):
    c = lax.axis_index("c")
    s = lax.axis_index("s")
    wid = c * NS + s
    pltpu.sync_copy(dst2_h.at[wid], dstv)
    _zero_vmem_1d(deg_v, NP)
    ones = jnp.ones((16,), jnp.float32)

    def step(i, carry):
        for j in range(KD // 16):
            di = dstv[i, pl.ds(j * 16, 16)]
            plsc.addupdate_scatter(deg_v, [di], ones)
        return carry

    lax.fori_loop(0, NCHD, step, 0)

    pltpu.sync_copy(deg_v, stage.at[s])
    plsc.subcore_barrier()
    pltpu.sync_copy(stage.at[:, pl.ds(s * RPT, RPT)], tmp_v)

    def red(w, carry):
        acc16 = tmp_v[0, pl.ds(w * 16, 16)]
        for t in range(1, NS):
            acc16 = acc16 + tmp_v[t, pl.ds(w * 16, 16)]
        out_v[pl.ds(w * 16, 16)] = acc16
        return carry

    lax.fori_loop(0, RPT // 16, red, 0)
    pltpu.sync_copy(out_v, out_h.at[c, pl.ds(s * RPT, RPT)])


def _run_deg(dst2):
    f = pl.kernel(
        _sc_deg,
        out_type=jax.ShapeDtypeStruct((NC, NP), jnp.float32),
        mesh=_mesh,
        compiler_params=_sc_params,
        scratch_types=[
            pltpu.VMEM((NCHD, KD), jnp.int32),
            pltpu.VMEM((NP,), jnp.float32),
            pltpu.VMEM_SHARED((NS, NP), jnp.float32),
            pltpu.VMEM((NS, RPT), jnp.float32),
            pltpu.VMEM((RPT,), jnp.float32),
            pltpu.SemaphoreType.DMA,
        ],
    )
    return f(dst2)


def _sc_spmm(width, src2_h, dst2_h, tbl_h, zeros_h, out_h,
             srcv, dstv, rows, acc, isems, gsems, ssems):
    c = lax.axis_index("c")
    s = lax.axis_index("s")
    wid = c * NS + s
    pltpu.sync_copy(zeros_h.at[pl.ds(s * RPT, RPT)], acc.at[pl.ds(s * RPT, RPT)])
    plsc.subcore_barrier()

    def issue_idx(ci):
        b = ci % DEPTH
        pltpu.async_copy(src2_h.at[wid, ci], srcv[b], isems[b])
        pltpu.async_copy(dst2_h.at[wid, ci], dstv[b], isems[b])

    def issue_gather(ci):
        b = ci % DEPTH
        pltpu.make_async_copy(src2_h.at[wid, 0], srcv[b], isems[b]).wait()
        pltpu.make_async_copy(dst2_h.at[wid, 0], dstv[b], isems[b]).wait()
        pltpu.async_copy(tbl_h.at[srcv[b]], rows[b], gsems[b])

    def issue_scatter(ci):
        b = ci % DEPTH
        pltpu.make_async_copy(tbl_h.at[srcv[b]], rows[b], gsems[b]).wait()
        pltpu.async_copy(rows[b], acc.at[dstv[b]], ssems[b], add=True)

    def wait_scatter(ci):
        b = ci % DEPTH
        pltpu.make_async_copy(rows[b], acc.at[dstv[b]], ssems[b]).wait()

    for t in range(NCHS + 4):
        c0 = t
        if c0 < NCHS:
            if c0 >= DEPTH:
                wait_scatter(c0 - DEPTH)
            issue_idx(c0)
        c1 = t - 2
        if 0 <= c1 < NCHS:
            issue_gather(c1)
        c2 = t - 4
        if 0 <= c2 < NCHS:
            issue_scatter(c2)
    for ci in range(NCHS - DEPTH, NCHS):
        wait_scatter(ci)
    plsc.subcore_barrier()
    pltpu.sync_copy(acc.at[pl.ds(s * RPT, RPT)], out_h.at[c, pl.ds(s * RPT, RPT)])


def _run_spmm(src2, dst2, table, zeros_w, width):
    f = pl.kernel(
        functools.partial(_sc_spmm, width),
        out_type=jax.ShapeDtypeStruct((NC, NP, width), jnp.float32),
        mesh=_mesh,
        compiler_params=_sc_params,
        scratch_types=[
            [pltpu.VMEM((KS,), jnp.int32) for _ in range(DEPTH)],
            [pltpu.VMEM((KS,), jnp.int32) for _ in range(DEPTH)],
            [pltpu.VMEM((KS, width), jnp.float32) for _ in range(DEPTH)],
            pltpu.VMEM_SHARED((NP, width), jnp.float32),
            [pltpu.SemaphoreType.DMA for _ in range(DEPTH)],
            [pltpu.SemaphoreType.DMA for _ in range(DEPTH)],
            [pltpu.SemaphoreType.DMA for _ in range(DEPTH)],
        ],
    )
    return f(src2, dst2, table, zeros_w)


def _sc_att(width, src2_h, dst2_h, sd_h, tbl_h, zeros_h, out_h,
            srcv, dstv, sds, sdd, exb, rows, acc, isems, hsems, gsems, ssems):
    wa = width + 16
    c = lax.axis_index("c")
    s = lax.axis_index("s")
    wid = c * NS + s
    pltpu.sync_copy(zeros_h.at[pl.ds(s * RPT, RPT)], acc.at[pl.ds(s * RPT, RPT)])
    plsc.subcore_barrier()

    iota16 = lax.iota(jnp.int32, 16)
    zero16 = jnp.zeros((16,), jnp.int32)
    one16 = jnp.ones((16,), jnp.int32)

    def issue_idx(b, ci):
        pltpu.async_copy(src2_h.at[wid, ci], srcv[b], isems[b])
        pltpu.async_copy(dst2_h.at[wid, ci], dstv[b], isems[b])

    def issue_gathers(b):
        pltpu.make_async_copy(src2_h.at[wid, 0], srcv[b], isems[b]).wait()
        pltpu.make_async_copy(dst2_h.at[wid, 0], dstv[b], isems[b]).wait()
        pltpu.async_copy(sd_h.at[srcv[b]], sds[b], hsems[b])
        pltpu.async_copy(sd_h.at[dstv[b]], sdd[b], hsems[b])
        pltpu.async_copy(tbl_h.at[srcv[b]], rows[b], gsems[b])

    def compute_and_scatter(b):
        pltpu.make_async_copy(sd_h.at[srcv[b]], sds[b], hsems[b]).wait()
        pltpu.make_async_copy(sd_h.at[dstv[b]], sdd[b], hsems[b]).wait()
        sval = plsc.load_gather(sds[b], [iota16, zero16])
        dval = plsc.load_gather(sdd[b], [iota16, one16])
        e = sval + dval
        e = jnp.maximum(e, 0.2 * e)
        exb[b][...] = jnp.exp(e)
        pltpu.make_async_copy(tbl_h.at[srcv[b]], rows[b], gsems[b]).wait()

        def scale(r, carry):
            cvec = plsc.load_gather(exb[b], [jnp.full((16,), r, jnp.int32)])
            for g in range(wa // 16):
                rows[b][r, pl.ds(g * 16, 16)] = (
                    rows[b][r, pl.ds(g * 16, 16)] * cvec)
            return carry

        lax.fori_loop(0, KA, scale, 0)
        pltpu.async_copy(rows[b], acc.at[dstv[b]], ssems[b], add=True)

    def wait_scatter(b):
        pltpu.make_async_copy(rows[b], acc.at[dstv[b]], ssems[b]).wait()

    for b in range(NB):
        issue_idx(b, b)

    def round_body(g, carry):
        for b in range(NB):
            issue_gathers(b)
        for b in range(NB):
            compute_and_scatter(b)
        for b in range(NB):
            wait_scatter(b)
            issue_idx(b, (g + 1) * NB + b)
        return carry

    lax.fori_loop(0, NROUND - 1, round_body, 0)
    for b in range(NB):
        issue_gathers(b)
    for b in range(NB):
        compute_and_scatter(b)
    for b in range(NB):
        wait_scatter(b)

    plsc.subcore_barrier()
    pltpu.sync_copy(acc.at[pl.ds(s * RPT, RPT)], out_h.at[c, pl.ds(s * RPT, RPT)])


def _run_att(src2, dst2, sd, table, zeros_wa, width):
    wa = width + 16
    f = pl.kernel(
        functools.partial(_sc_att, width),
        out_type=jax.ShapeDtypeStruct((NC, NP, wa), jnp.float32),
        mesh=_mesh,
        compiler_params=_sc_params,
        scratch_types=[
            [pltpu.VMEM((KA,), jnp.int32) for _ in range(NB)],
            [pltpu.VMEM((KA,), jnp.int32) for _ in range(NB)],
            [pltpu.VMEM((KA, 16), jnp.float32) for _ in range(NB)],
            [pltpu.VMEM((KA, 16), jnp.float32) for _ in range(NB)],
            [pltpu.VMEM((16,), jnp.float32) for _ in range(NB)],
            [pltpu.VMEM((KA, wa), jnp.float32) for _ in range(NB)],
            pltpu.VMEM_SHARED((NP, wa), jnp.float32),
            [pltpu.SemaphoreType.DMA for _ in range(NB)],
            [pltpu.SemaphoreType.DMA for _ in range(NB)],
            [pltpu.SemaphoreType.DMA for _ in range(NB)],
            [pltpu.SemaphoreType.DMA for _ in range(NB)],
        ],
    )
    return f(src2, dst2, sd, table, zeros_wa)


BN = 1000
GRID = N // BN


def _row_spec(width):
    return pl.BlockSpec((BN, width), lambda i: (i, 0))


def _full_spec(shape):
    nd = len(shape)
    return pl.BlockSpec(shape, lambda i: (0,) * nd)


def _rsq_from(deg0_ref, deg1_ref):
    deg = jnp.maximum(deg0_ref[...] + deg1_ref[...], 1.0)
    return lax.rsqrt(deg)


def _ones_cols(base, width):
    bn = base.shape[0]
    return jnp.concatenate(
        [base, jnp.ones((bn, 1), jnp.float32),
         jnp.zeros((bn, 15), jnp.float32)], axis=1)


def _tc1(x_ref, wg1_ref, wx1_ref, bx1_ref, deg0_ref, deg1_ref,
         t1s_ref, hx1_ref):
    xb = x_ref[...]
    rsq = _rsq_from(deg0_ref, deg1_ref)
    t1s_ref[...] = jnp.dot(xb, wg1_ref[...],
                           preferred_element_type=jnp.float32) * rsq
    hx1 = jnp.dot(xb, wx1_ref[...],
                  preferred_element_type=jnp.float32) + bx1_ref[...]
    hx1_ref[...] = _ones_cols(hx1, 128)


def _tc2(p0_ref, p1_ref, bg1_ref, wg2_ref, deg0_ref, deg1_ref, t2s_ref):
    rsq = _rsq_from(deg0_ref, deg1_ref)
    h1 = jnp.maximum((p0_ref[...] + p1_ref[...]) * rsq + bg1_ref[...], 0.0)
    t2s_ref[...] = jnp.dot(h1, wg2_ref[...],
                           preferred_element_type=jnp.float32) * rsq


def _tc3(q0_ref, q1_ref, bg2_ref, wy1_ref, by1_ref, a1_ref,
         deg0_ref, deg1_ref, z_ref, hy1s_ref, sd1_ref):
    rsq = _rsq_from(deg0_ref, deg1_ref)
    z = (q0_ref[...] + q1_ref[...]) * rsq + bg2_ref[...]
    z_ref[...] = z
    zs = z - jnp.max(z, axis=-1, keepdims=True)
    ez = jnp.exp(zs)
    preds = ez / jnp.sum(ez, axis=-1, keepdims=True)
    hy1 = jnp.dot(preds, wy1_ref[...],
                  preferred_element_type=jnp.float32) + by1_ref[...]
    sd1_ref[...] = jnp.dot(hy1, a1_ref[...], preferred_element_type=jnp.float32)
    hy1s_ref[...] = hy1 * rsq


def _tc4(r0_ref, r1_ref, d10_ref, d11_ref, u0_ref, u1_ref,
         wx2_ref, bx2_ref, wy2_ref, by2_ref, a2_ref, deg0_ref, deg1_ref,
         hx2_ref, sd2_ref):
    rsq = _rsq_from(deg0_ref, deg1_ref)
    invd = 1.0 / (d10_ref[...] + d11_ref[...] + 1e-16)
    x1 = jnp.maximum((r0_ref[...] + r1_ref[...]) * invd, 0.0)
    hx2 = jnp.dot(x1, wx2_ref[...],
                  preferred_element_type=jnp.float32) + bx2_ref[...]
    hx2_ref[...] = _ones_cols(hx2, 64)
    y1 = jnp.maximum((u0_ref[...] + u1_ref[...]) * rsq, 0.0)
    hy2 = jnp.dot(y1, wy2_ref[...],
                  preferred_element_type=jnp.float32) + by2_ref[...]
    sd2_ref[...] = jnp.dot(hy2, a2_ref[...], preferred_element_type=jnp.float32)


def _tc5(z_ref, v0_ref, v1_ref, d20_ref, d21_ref, out_ref):
    invd = 1.0 / (d20_ref[...] + d21_ref[...] + 1e-16)
    out_ref[...] = (z_ref[...] + (v0_ref[...] + v1_ref[...]) * invd) * 0.5


def _sd_matrix(a_src, a_dst):
    return jnp.concatenate(
        [a_src.reshape(64, 1), a_dst.reshape(64, 1),
         jnp.zeros((64, 14), jnp.float32)], axis=1)


def kernel(x, edge_index, W_g1, b_g1, W_g2, b_g2, W_x1, b_x1, W_y1, b_y1,
           a_src1, a_dst1, W_x2, b_x2, W_y2, b_y2, a_src2, a_dst2):
    f32 = jnp.float32
    src = edge_index[0]
    dst = edge_index[1]
    srcD = src.reshape(NW, NCHD, KD)
    dstD = dst.reshape(NW, NCHD, KD)
    srcS = src.reshape(NW, NCHS, KS)
    dstS = dst.reshape(NW, NCHS, KS)
    srcA = src.reshape(NW, NCHA, KA)
    dstA = dst.reshape(NW, NCHA, KA)

    zeros64 = jnp.zeros((NP, 64), f32)
    zeros80 = jnp.zeros((NP, 80), f32)
    zeros128 = jnp.zeros((NP, 128), f32)
    zeros144 = jnp.zeros((NP, 144), f32)

    degp = _run_deg(dstD)
    deg0 = degp[0, :N].reshape(N, 1)
    deg1 = degp[1, :N].reshape(N, 1)

    t1s, hx1 = pl.pallas_call(
        _tc1,
        grid=(GRID,),
        in_specs=[
            _row_spec(128), _full_spec((128, 128)), _full_spec((128, 128)),
            _full_spec((1, 128)), _row_spec(1), _row_spec(1),
        ],
        out_specs=[_row_spec(128), _row_spec(144)],
        out_shape=[
            jax.ShapeDtypeStruct((N, 128), f32),
            jax.ShapeDtypeStruct((N, 144), f32),
        ],
    )(x, W_g1, W_x1, b_x1.reshape(1, 128), deg0, deg1)

    p = _run_spmm(srcS, dstS, t1s, zeros128, 128)

    t2s = pl.pallas_call(
        _tc2,
        grid=(GRID,),
        in_specs=[
            _row_spec(128), _row_spec(128), _full_spec((1, 128)),
            _full_spec((128, 64)), _row_spec(1), _row_spec(1),
        ],
        out_specs=_row_spec(64),
        out_shape=jax.ShapeDtypeStruct((N, 64), f32),
    )(p[0, :N], p[1, :N], b_g1.reshape(1, 128), W_g2, deg0, deg1)

    q = _run_spmm(srcS, dstS, t2s, zeros64, 64)

    a1 = _sd_matrix(a_src1, a_dst1)
    z, hy1s, sd1 = pl.pallas_call(
        _tc3,
        grid=(GRID,),
        in_specs=[
            _row_spec(64), _row_spec(64), _full_spec((1, 64)),
            _full_spec((64, 64)), _full_spec((1, 64)), _full_spec((64, 16)),
            _row_spec(1), _row_spec(1),
        ],
        out_specs=[_row_spec(64), _row_spec(64), _row_spec(16)],
        out_shape=[
            jax.ShapeDtypeStruct((N, 64), f32),
            jax.ShapeDtypeStruct((N, 64), f32),
            jax.ShapeDtypeStruct((N, 16), f32),
        ],
    )(q[0, :N], q[1, :N], b_g2.reshape(1, 64), W_y1, b_y1.reshape(1, 64), a1,
      deg0, deg1)

    r = _run_att(srcA, dstA, sd1, hx1, zeros144, 128)
    u = _run_spmm(srcS, dstS, hy1s, zeros64, 64)

    r0 = r[0, :N, :128]
    r1 = r[1, :N, :128]
    den10 = r[0, :N, 128].reshape(N, 1)
    den11 = r[1, :N, 128].reshape(N, 1)

    a2 = _sd_matrix(a_src2, a_dst2)
    hx2, sd2 = pl.pallas_call(
        _tc4,
        grid=(GRID,),
        in_specs=[
            _row_spec(128), _row_spec(128), _row_spec(1), _row_spec(1),
            _row_spec(64), _row_spec(64),
            _full_spec((128, 64)), _full_spec((1, 64)),
            _full_spec((64, 64)), _full_spec((1, 64)), _full_spec((64, 16)),
            _row_spec(1), _row_spec(1),
        ],
        out_specs=[_row_spec(80), _row_spec(16)],
        out_shape=[
            jax.ShapeDtypeStruct((N, 80), f32),
            jax.ShapeDtypeStruct((N, 16), f32),
        ],
    )(r0, r1, den10, den11, u[0, :N], u[1, :N],
      W_x2, b_x2.reshape(1, 64), W_y2, b_y2.reshape(1, 64), a2, deg0, deg1)

    v = _run_att(srcA, dstA, sd2, hx2, zeros80, 64)

    v0 = v[0, :N, :64]
    v1 = v[1, :N, :64]
    den20 = v[0, :N, 64].reshape(N, 1)
    den21 = v[1, :N, 64].reshape(N, 1)

    out = pl.pallas_call(
        _tc5,
        grid=(GRID,),
        in_specs=[
            _row_spec(64), _row_spec(64), _row_spec(64),
            _row_spec(1), _row_spec(1),
        ],
        out_specs=_row_spec(64),
        out_shape=jax.ShapeDtypeStruct((N, 64), f32),
    )(z, v0, v1, den20, den21)
    return out


# speedup vs baseline: 35.1469x; 1.2773x over previous
import functools

import jax
import jax.numpy as jnp
from jax import lax
from jax.experimental import pallas as pl
from jax.experimental.pallas import tpu as pltpu
from jax.experimental.pallas import tpu_sc as plsc

N = 10000
NP = 10240
E = 320000

NC = 2
NS = 16
NW = NC * NS
EPT = E // NW
RPT = NP // NS

KD = 80
NCHD = EPT // KD
KS128 = 40
KS64 = 80
DEPTH = 6
KA128 = 40
KA64 = 80
NB = 5

_mesh = plsc.VectorSubcoreMesh(
    core_axis_name="c", subcore_axis_name="s", num_cores=NC, num_subcores=NS)

_sc_params = pltpu.CompilerParams(
    use_tc_tiling_on_sc=False, needs_layout_passes=False)


def _zero_vmem_1d(ref, n):
    zv = jnp.zeros((16,), jnp.float32)

    def body(i, carry):
        ref[pl.ds(i * 16, 16)] = zv
        return carry

    lax.fori_loop(0, n // 16, body, 0)


def _sc_deg(dst2_h, out_h, dstv, deg_v, stage, tmp_v, out_v, sem):
    c = lax.axis_index("c")
    s = lax.axis_index("s")
    wid = c * NS + s
    pltpu.sync_copy(dst2_h.at[wid], dstv)
    _zero_vmem_1d(deg_v, NP)
    ones = jnp.ones((16,), jnp.float32)

    def step(i, carry):
        for j in range(KD // 16):
            di = dstv[i, pl.ds(j * 16, 16)]
            plsc.addupdate_scatter(deg_v, [di], ones)
        return carry

    lax.fori_loop(0, NCHD, step, 0)

    pltpu.sync_copy(deg_v, stage.at[s])
    plsc.subcore_barrier()
    pltpu.sync_copy(stage.at[:, pl.ds(s * RPT, RPT)], tmp_v)

    def red(w, carry):
        acc16 = tmp_v[0, pl.ds(w * 16, 16)]
        for t in range(1, NS):
            acc16 = acc16 + tmp_v[t, pl.ds(w * 16, 16)]
        out_v[pl.ds(w * 16, 16)] = acc16
        return carry

    lax.fori_loop(0, RPT // 16, red, 0)
    pltpu.sync_copy(out_v, out_h.at[c, pl.ds(s * RPT, RPT)])


def _run_deg(dst2):
    f = pl.kernel(
        _sc_deg,
        out_type=jax.ShapeDtypeStruct((NC, NP), jnp.float32),
        mesh=_mesh,
        compiler_params=_sc_params,
        scratch_types=[
            pltpu.VMEM((NCHD, KD), jnp.int32),
            pltpu.VMEM((NP,), jnp.float32),
            pltpu.VMEM_SHARED((NS, NP), jnp.float32),
            pltpu.VMEM((NS, RPT), jnp.float32),
            pltpu.VMEM((RPT,), jnp.float32),
            pltpu.SemaphoreType.DMA,
        ],
    )
    return f(dst2)


def _sc_spmm(width, ks, src2_h, dst2_h, tbl_h, zeros_h, out_h,
             srcv, dstv, rows, acc, isems, gsems, ssems):
    c = lax.axis_index("c")
    s = lax.axis_index("s")
    wid = c * NS + s
    pltpu.sync_copy(zeros_h.at[pl.ds(s * RPT, RPT)], acc.at[pl.ds(s * RPT, RPT)])
    plsc.subcore_barrier()

    def issue_idx(ci):
        b = ci % DEPTH
        pltpu.async_copy(src2_h.at[wid, ci], srcv[b], isems[b])
        pltpu.async_copy(dst2_h.at[wid, ci], dstv[b], isems[b])

    def issue_gather(ci):
        b = ci % DEPTH
        pltpu.make_async_copy(src2_h.at[wid, 0], srcv[b], isems[b]).wait()
        pltpu.make_async_copy(dst2_h.at[wid, 0], dstv[b], isems[b]).wait()
        pltpu.async_copy(tbl_h.at[srcv[b]], rows[b], gsems[b])

    def issue_scatter(ci):
        b = ci % DEPTH
        pltpu.make_async_copy(tbl_h.at[srcv[b]], rows[b], gsems[b]).wait()
        pltpu.async_copy(rows[b], acc.at[dstv[b]], ssems[b], add=True)

    def wait_scatter(ci):
        b = ci % DEPTH
        pltpu.make_async_copy(rows[b], acc.at[dstv[b]], ssems[b]).wait()

    nchs = EPT // ks
    for t in range(nchs + 4):
        c0 = t
        if c0 < nchs:
            if c0 >= DEPTH:
                wait_scatter(c0 - DEPTH)
            issue_idx(c0)
        c1 = t - 2
        if 0 <= c1 < nchs:
            issue_gather(c1)
        c2 = t - 4
        if 0 <= c2 < nchs:
            issue_scatter(c2)
    for ci in range(nchs - DEPTH, nchs):
        wait_scatter(ci)
    plsc.subcore_barrier()
    pltpu.sync_copy(acc.at[pl.ds(s * RPT, RPT)], out_h.at[c, pl.ds(s * RPT, RPT)])


def _run_spmm(src2, dst2, table, zeros_w, width, ks):
    f = pl.kernel(
        functools.partial(_sc_spmm, width, ks),
        out_type=jax.ShapeDtypeStruct((NC, NP, width), jnp.float32),
        mesh=_mesh,
        compiler_params=_sc_params,
        scratch_types=[
            [pltpu.VMEM((ks,), jnp.int32) for _ in range(DEPTH)],
            [pltpu.VMEM((ks,), jnp.int32) for _ in range(DEPTH)],
            [pltpu.VMEM((ks, width), jnp.float32) for _ in range(DEPTH)],
            pltpu.VMEM_SHARED((NP, width), jnp.float32),
            [pltpu.SemaphoreType.DMA for _ in range(DEPTH)],
            [pltpu.SemaphoreType.DMA for _ in range(DEPTH)],
            [pltpu.SemaphoreType.DMA for _ in range(DEPTH)],
        ],
    )
    return f(src2, dst2, table, zeros_w)


def _sc_att(width, ka, src2_h, dst2_h, sd_h, tbl_h, zeros_h, out_h,
            srcv, dstv, sds, sdd, exb, rows, acc, isems, hsems, gsems, ssems):
    wa = width + 16
    c = lax.axis_index("c")
    s = lax.axis_index("s")
    wid = c * NS + s
    pltpu.sync_copy(zeros_h.at[pl.ds(s * RPT, RPT)], acc.at[pl.ds(s * RPT, RPT)])
    plsc.subcore_barrier()

    iota16 = lax.iota(jnp.int32, 16)
    zero16 = jnp.zeros((16,), jnp.int32)
    one16 = jnp.ones((16,), jnp.int32)

    def issue_idx(b, ci):
        pltpu.async_copy(src2_h.at[wid, ci], srcv[b], isems[b])
        pltpu.async_copy(dst2_h.at[wid, ci], dstv[b], isems[b])

    def issue_gathers(b):
        pltpu.make_async_copy(src2_h.at[wid, 0], srcv[b], isems[b]).wait()
        pltpu.make_async_copy(dst2_h.at[wid, 0], dstv[b], isems[b]).wait()
        pltpu.async_copy(sd_h.at[srcv[b]], sds[b], hsems[b])
        pltpu.async_copy(sd_h.at[dstv[b]], sdd[b], hsems[b])
        pltpu.async_copy(tbl_h.at[srcv[b]], rows[b], gsems[b])

    def compute_and_scatter(b):
        pltpu.make_async_copy(sd_h.at[srcv[b]], sds[b], hsems[b]).wait()
        pltpu.make_async_copy(sd_h.at[dstv[b]], sdd[b], hsems[b]).wait()
        offs = list(range(0, ka - 15, 16))
        if ka % 16:
            offs.append(ka - 16)
        for off in offs:
            ridx = iota16 + off
            sval = plsc.load_gather(sds[b], [ridx, zero16])
            dval = plsc.load_gather(sdd[b], [ridx, one16])
            e = sval + dval
            e = jnp.maximum(e, 0.2 * e)
            exb[b][pl.ds(off, 16)] = jnp.exp(e)
        pltpu.make_async_copy(tbl_h.at[srcv[b]], rows[b], gsems[b]).wait()

        def scale(r, carry):
            cvec = plsc.load_gather(exb[b], [jnp.full((16,), r, jnp.int32)])
            for g in range(wa // 16):
                rows[b][r, pl.ds(g * 16, 16)] = (
                    rows[b][r, pl.ds(g * 16, 16)] * cvec)
            return carry

        lax.fori_loop(0, ka, scale, 0)
        pltpu.async_copy(rows[b], acc.at[dstv[b]], ssems[b], add=True)

    def wait_scatter(b):
        pltpu.make_async_copy(rows[b], acc.at[dstv[b]], ssems[b]).wait()

    for b in range(NB):
        issue_idx(b, b)

    def round_body(g, carry):
        for b in range(NB):
            issue_gathers(b)
        for b in range(NB):
            compute_and_scatter(b)
        for b in range(NB):
            wait_scatter(b)
            issue_idx(b, (g + 1) * NB + b)
        return carry

    nround = EPT // ka // NB
    lax.fori_loop(0, nround - 1, round_body, 0)
    for b in range(NB):
        issue_gathers(b)
    for b in range(NB):
        compute_and_scatter(b)
    for b in range(NB):
        wait_scatter(b)

    plsc.subcore_barrier()
    pltpu.sync_copy(acc.at[pl.ds(s * RPT, RPT)], out_h.at[c, pl.ds(s * RPT, RPT)])


def _run_att(src2, dst2, sd, table, zeros_wa, width, ka):
    wa = width + 16
    f = pl.kernel(
        functools.partial(_sc_att, width, ka),
        out_type=jax.ShapeDtypeStruct((NC, NP, wa), jnp.float32),
        mesh=_mesh,
        compiler_params=_sc_params,
        scratch_types=[
            [pltpu.VMEM((ka,), jnp.int32) for _ in range(NB)],
            [pltpu.VMEM((ka,), jnp.int32) for _ in range(NB)],
            [pltpu.VMEM((ka, 16), jnp.float32) for _ in range(NB)],
            [pltpu.VMEM((ka, 16), jnp.float32) for _ in range(NB)],
            [pltpu.VMEM((ka,), jnp.float32) for _ in range(NB)],
            [pltpu.VMEM((ka, wa), jnp.float32) for _ in range(NB)],
            pltpu.VMEM_SHARED((NP, wa), jnp.float32),
            [pltpu.SemaphoreType.DMA for _ in range(NB)],
            [pltpu.SemaphoreType.DMA for _ in range(NB)],
            [pltpu.SemaphoreType.DMA for _ in range(NB)],
            [pltpu.SemaphoreType.DMA for _ in range(NB)],
        ],
    )
    return f(src2, dst2, sd, table, zeros_wa)


BN = 1000
GRID = N // BN


def _row_spec(width):
    return pl.BlockSpec((BN, width), lambda i: (i, 0))


def _full_spec(shape):
    nd = len(shape)
    return pl.BlockSpec(shape, lambda i: (0,) * nd)


def _rsq_from(deg0_ref, deg1_ref):
    deg = jnp.maximum(deg0_ref[...] + deg1_ref[...], 1.0)
    return lax.rsqrt(deg)


def _ones_cols(base, width):
    bn = base.shape[0]
    return jnp.concatenate(
        [base, jnp.ones((bn, 1), jnp.float32),
         jnp.zeros((bn, 15), jnp.float32)], axis=1)


def _tc1(x_ref, wg1_ref, wx1_ref, bx1_ref, deg0_ref, deg1_ref,
         t1s_ref, hx1_ref):
    xb = x_ref[...]
    rsq = _rsq_from(deg0_ref, deg1_ref)
    t1s_ref[...] = jnp.dot(xb, wg1_ref[...],
                           preferred_element_type=jnp.float32) * rsq
    hx1 = jnp.dot(xb, wx1_ref[...],
                  preferred_element_type=jnp.float32) + bx1_ref[...]
    hx1_ref[...] = _ones_cols(hx1, 128)


def _tc2(p0_ref, p1_ref, bg1_ref, wg2_ref, deg0_ref, deg1_ref, t2s_ref):
    rsq = _rsq_from(deg0_ref, deg1_ref)
    h1 = jnp.maximum((p0_ref[...] + p1_ref[...]) * rsq + bg1_ref[...], 0.0)
    t2s_ref[...] = jnp.dot(h1, wg2_ref[...],
                           preferred_element_type=jnp.float32) * rsq


def _tc3(q0_ref, q1_ref, bg2_ref, wy1_ref, by1_ref, a1_ref,
         deg0_ref, deg1_ref, z_ref, hy1s_ref, sd1_ref):
    rsq = _rsq_from(deg0_ref, deg1_ref)
    z = (q0_ref[...] + q1_ref[...]) * rsq + bg2_ref[...]
    z_ref[...] = z
    zs = z - jnp.max(z, axis=-1, keepdims=True)
    ez = jnp.exp(zs)
    preds = ez / jnp.sum(ez, axis=-1, keepdims=True)
    hy1 = jnp.dot(preds, wy1_ref[...],
                  preferred_element_type=jnp.float32) + by1_ref[...]
    sd1_ref[...] = jnp.dot(hy1, a1_ref[...], preferred_element_type=jnp.float32)
    hy1s_ref[...] = hy1 * rsq


def _tc4(r0_ref, r1_ref, d10_ref, d11_ref, u0_ref, u1_ref,
         wx2_ref, bx2_ref, wy2_ref, by2_ref, a2_ref, deg0_ref, deg1_ref,
         hx2_ref, sd2_ref):
    rsq = _rsq_from(deg0_ref, deg1_ref)
    invd = 1.0 / (d10_ref[...] + d11_ref[...] + 1e-16)
    x1 = jnp.maximum((r0_ref[...] + r1_ref[...]) * invd, 0.0)
    hx2 = jnp.dot(x1, wx2_ref[...],
                  preferred_element_type=jnp.float32) + bx2_ref[...]
    hx2_ref[...] = _ones_cols(hx2, 64)
    y1 = jnp.maximum((u0_ref[...] + u1_ref[...]) * rsq, 0.0)
    hy2 = jnp.dot(y1, wy2_ref[...],
                  preferred_element_type=jnp.float32) + by2_ref[...]
    sd2_ref[...] = jnp.dot(hy2, a2_ref[...], preferred_element_type=jnp.float32)


def _tc5(z_ref, v0_ref, v1_ref, d20_ref, d21_ref, out_ref):
    invd = 1.0 / (d20_ref[...] + d21_ref[...] + 1e-16)
    out_ref[...] = (z_ref[...] + (v0_ref[...] + v1_ref[...]) * invd) * 0.5


def _sd_matrix(a_src, a_dst):
    return jnp.concatenate(
        [a_src.reshape(64, 1), a_dst.reshape(64, 1),
         jnp.zeros((64, 14), jnp.float32)], axis=1)


def kernel(x, edge_index, W_g1, b_g1, W_g2, b_g2, W_x1, b_x1, W_y1, b_y1,
           a_src1, a_dst1, W_x2, b_x2, W_y2, b_y2, a_src2, a_dst2):
    f32 = jnp.float32
    src = edge_index[0]
    dst = edge_index[1]
    srcD = src.reshape(NW, NCHD, KD)
    dstD = dst.reshape(NW, NCHD, KD)
    srcS1 = src.reshape(NW, EPT // KS128, KS128)
    dstS1 = dst.reshape(NW, EPT // KS128, KS128)
    srcS2 = src.reshape(NW, EPT // KS64, KS64)
    dstS2 = dst.reshape(NW, EPT // KS64, KS64)
    srcA1 = src.reshape(NW, EPT // KA128, KA128)
    dstA1 = dst.reshape(NW, EPT // KA128, KA128)
    srcA2 = src.reshape(NW, EPT // KA64, KA64)
    dstA2 = dst.reshape(NW, EPT // KA64, KA64)

    zeros64 = jnp.zeros((NP, 64), f32)
    zeros80 = jnp.zeros((NP, 80), f32)
    zeros128 = jnp.zeros((NP, 128), f32)
    zeros144 = jnp.zeros((NP, 144), f32)

    degp = _run_deg(dstD)
    deg0 = degp[0, :N].reshape(N, 1)
    deg1 = degp[1, :N].reshape(N, 1)

    t1s, hx1 = pl.pallas_call(
        _tc1,
        grid=(GRID,),
        in_specs=[
            _row_spec(128), _full_spec((128, 128)), _full_spec((128, 128)),
            _full_spec((1, 128)), _row_spec(1), _row_spec(1),
        ],
        out_specs=[_row_spec(128), _row_spec(144)],
        out_shape=[
            jax.ShapeDtypeStruct((N, 128), f32),
            jax.ShapeDtypeStruct((N, 144), f32),
        ],
    )(x, W_g1, W_x1, b_x1.reshape(1, 128), deg0, deg1)

    p = _run_spmm(srcS1, dstS1, t1s, zeros128, 128, KS128)

    t2s = pl.pallas_call(
        _tc2,
        grid=(GRID,),
        in_specs=[
            _row_spec(128), _row_spec(128), _full_spec((1, 128)),
            _full_spec((128, 64)), _row_spec(1), _row_spec(1),
        ],
        out_specs=_row_spec(64),
        out_shape=jax.ShapeDtypeStruct((N, 64), f32),
    )(p[0, :N], p[1, :N], b_g1.reshape(1, 128), W_g2, deg0, deg1)

    q = _run_spmm(srcS2, dstS2, t2s, zeros64, 64, KS64)

    a1 = _sd_matrix(a_src1, a_dst1)
    z, hy1s, sd1 = pl.pallas_call(
        _tc3,
        grid=(GRID,),
        in_specs=[
            _row_spec(64), _row_spec(64), _full_spec((1, 64)),
            _full_spec((64, 64)), _full_spec((1, 64)), _full_spec((64, 16)),
            _row_spec(1), _row_spec(1),
        ],
        out_specs=[_row_spec(64), _row_spec(64), _row_spec(16)],
        out_shape=[
            jax.ShapeDtypeStruct((N, 64), f32),
            jax.ShapeDtypeStruct((N, 64), f32),
            jax.ShapeDtypeStruct((N, 16), f32),
        ],
    )(q[0, :N], q[1, :N], b_g2.reshape(1, 64), W_y1, b_y1.reshape(1, 64), a1,
      deg0, deg1)

    r = _run_att(srcA1, dstA1, sd1, hx1, zeros144, 128, KA128)
    u = _run_spmm(srcS2, dstS2, hy1s, zeros64, 64, KS64)

    r0 = r[0, :N, :128]
    r1 = r[1, :N, :128]
    den10 = r[0, :N, 128].reshape(N, 1)
    den11 = r[1, :N, 128].reshape(N, 1)

    a2 = _sd_matrix(a_src2, a_dst2)
    hx2, sd2 = pl.pallas_call(
        _tc4,
        grid=(GRID,),
        in_specs=[
            _row_spec(128), _row_spec(128), _row_spec(1), _row_spec(1),
            _row_spec(64), _row_spec(64),
            _full_spec((128, 64)), _full_spec((1, 64)),
            _full_spec((64, 64)), _full_spec((1, 64)), _full_spec((64, 16)),
            _row_spec(1), _row_spec(1),
        ],
        out_specs=[_row_spec(80), _row_spec(16)],
        out_shape=[
            jax.ShapeDtypeStruct((N, 80), f32),
            jax.ShapeDtypeStruct((N, 16), f32),
        ],
    )(r0, r1, den10, den11, u[0, :N], u[1, :N],
      W_x2, b_x2.reshape(1, 64), W_y2, b_y2.reshape(1, 64), a2, deg0, deg1)

    v = _run_att(srcA2, dstA2, sd2, hx2, zeros80, 64, KA64)

    v0 = v[0, :N, :64]
    v1 = v[1, :N, :64]
    den20 = v[0, :N, 64].reshape(N, 1)
    den21 = v[1, :N, 64].reshape(N, 1)

    out = pl.pallas_call(
        _tc5,
        grid=(GRID,),
        in_specs=[
            _row_spec(64), _row_spec(64), _row_spec(64),
            _row_spec(1), _row_spec(1),
        ],
        out_specs=_row_spec(64),
        out_shape=jax.ShapeDtypeStruct((N, 64), f32),
    )(z, v0, v1, den20, den21)
    return out


# speedup vs baseline: 36.1347x; 1.0281x over previous
import functools

import jax
import jax.numpy as jnp
from jax import lax
from jax.experimental import pallas as pl
from jax.experimental.pallas import tpu as pltpu
from jax.experimental.pallas import tpu_sc as plsc

N = 10000
NP = 10240
E = 320000

NC = 2
NS = 16
NW = NC * NS
EPT = E // NW
RPT = NP // NS

KD = 80
NCHD = EPT // KD
KS128 = 40
KS64 = 80
DEPTH = 8
KA128 = 40
KA64 = 80
NB = 5

_mesh = plsc.VectorSubcoreMesh(
    core_axis_name="c", subcore_axis_name="s", num_cores=NC, num_subcores=NS)

_sc_params = pltpu.CompilerParams(
    use_tc_tiling_on_sc=False, needs_layout_passes=False)


def _zero_vmem_1d(ref, n):
    zv = jnp.zeros((16,), jnp.float32)

    def body(i, carry):
        ref[pl.ds(i * 16, 16)] = zv
        return carry

    lax.fori_loop(0, n // 16, body, 0)


def _sc_deg(dst2_h, out_h, dstv, deg_v, stage, tmp_v, out_v, sem):
    c = lax.axis_index("c")
    s = lax.axis_index("s")
    wid = c * NS + s
    pltpu.sync_copy(dst2_h.at[wid], dstv)
    _zero_vmem_1d(deg_v, NP)
    ones = jnp.ones((16,), jnp.float32)

    def step(i, carry):
        for j in range(KD // 16):
            di = dstv[i, pl.ds(j * 16, 16)]
            plsc.addupdate_scatter(deg_v, [di], ones)
        return carry

    lax.fori_loop(0, NCHD, step, 0)

    pltpu.sync_copy(deg_v, stage.at[s])
    plsc.subcore_barrier()
    pltpu.sync_copy(stage.at[:, pl.ds(s * RPT, RPT)], tmp_v)

    def red(w, carry):
        acc16 = tmp_v[0, pl.ds(w * 16, 16)]
        for t in range(1, NS):
            acc16 = acc16 + tmp_v[t, pl.ds(w * 16, 16)]
        out_v[pl.ds(w * 16, 16)] = acc16
        return carry

    lax.fori_loop(0, RPT // 16, red, 0)
    pltpu.sync_copy(out_v, out_h.at[c, pl.ds(s * RPT, RPT)])


def _run_deg(dst2):
    f = pl.kernel(
        _sc_deg,
        out_type=jax.ShapeDtypeStruct((NC, NP), jnp.float32),
        mesh=_mesh,
        compiler_params=_sc_params,
        scratch_types=[
            pltpu.VMEM((NCHD, KD), jnp.int32),
            pltpu.VMEM((NP,), jnp.float32),
            pltpu.VMEM_SHARED((NS, NP), jnp.float32),
            pltpu.VMEM((NS, RPT), jnp.float32),
            pltpu.VMEM((RPT,), jnp.float32),
            pltpu.SemaphoreType.DMA,
        ],
    )
    return f(dst2)


def _sc_spmm(width, ks, src2_h, dst2_h, tbl_h, zeros_h, out_h,
             srcv, dstv, rows, acc, isems, gsems, ssems):
    c = lax.axis_index("c")
    s = lax.axis_index("s")
    wid = c * NS + s
    pltpu.sync_copy(zeros_h.at[pl.ds(s * RPT, RPT)], acc.at[pl.ds(s * RPT, RPT)])
    plsc.subcore_barrier()

    def issue_idx(ci):
        b = ci % DEPTH
        pltpu.async_copy(src2_h.at[wid, ci], srcv[b], isems[b])
        pltpu.async_copy(dst2_h.at[wid, ci], dstv[b], isems[b])

    def issue_gather(ci):
        b = ci % DEPTH
        pltpu.make_async_copy(src2_h.at[wid, 0], srcv[b], isems[b]).wait()
        pltpu.make_async_copy(dst2_h.at[wid, 0], dstv[b], isems[b]).wait()
        pltpu.async_copy(tbl_h.at[srcv[b]], rows[b], gsems[b])

    def issue_scatter(ci):
        b = ci % DEPTH
        pltpu.make_async_copy(tbl_h.at[srcv[b]], rows[b], gsems[b]).wait()
        pltpu.async_copy(rows[b], acc.at[dstv[b]], ssems[b], add=True)

    def wait_scatter(ci):
        b = ci % DEPTH
        pltpu.make_async_copy(rows[b], acc.at[dstv[b]], ssems[b]).wait()

    nchs = EPT // ks
    for t in range(nchs + 5):
        c0 = t
        if c0 < nchs:
            if c0 >= DEPTH:
                wait_scatter(c0 - DEPTH)
            issue_idx(c0)
        c1 = t - 2
        if 0 <= c1 < nchs:
            issue_gather(c1)
        c2 = t - 5
        if 0 <= c2 < nchs:
            issue_scatter(c2)
    for ci in range(nchs - DEPTH, nchs):
        wait_scatter(ci)
    plsc.subcore_barrier()
    pltpu.sync_copy(acc.at[pl.ds(s * RPT, RPT)], out_h.at[c, pl.ds(s * RPT, RPT)])


def _run_spmm(src2, dst2, table, zeros_w, width, ks):
    f = pl.kernel(
        functools.partial(_sc_spmm, width, ks),
        out_type=jax.ShapeDtypeStruct((NC, NP, width), jnp.float32),
        mesh=_mesh,
        compiler_params=_sc_params,
        scratch_types=[
            [pltpu.VMEM((ks,), jnp.int32) for _ in range(DEPTH)],
            [pltpu.VMEM((ks,), jnp.int32) for _ in range(DEPTH)],
            [pltpu.VMEM((ks, width), jnp.float32) for _ in range(DEPTH)],
            pltpu.VMEM_SHARED((NP, width), jnp.float32),
            [pltpu.SemaphoreType.DMA for _ in range(DEPTH)],
            [pltpu.SemaphoreType.DMA for _ in range(DEPTH)],
            [pltpu.SemaphoreType.DMA for _ in range(DEPTH)],
        ],
    )
    return f(src2, dst2, table, zeros_w)


def _sc_att(width, ka, src2_h, dst2_h, sd_h, tbl_h, zeros_h, out_h,
            srcv0, dstv0, srcv1, dstv1, sds, sdd, exb, rows, acc,
            isems0, isems1, hsems, gsems, ssems):
    wa = width + 16
    c = lax.axis_index("c")
    s = lax.axis_index("s")
    wid = c * NS + s
    pltpu.sync_copy(zeros_h.at[pl.ds(s * RPT, RPT)], acc.at[pl.ds(s * RPT, RPT)])
    plsc.subcore_barrier()

    iota16 = lax.iota(jnp.int32, 16)
    zero16 = jnp.zeros((16,), jnp.int32)
    one16 = jnp.ones((16,), jnp.int32)
    sets = ((srcv0, dstv0, isems0), (srcv1, dstv1, isems1))

    def issue_idx(cur, b, ci):
        sv, dv, isem = sets[cur]
        pltpu.async_copy(src2_h.at[wid, ci], sv[b], isem[b])
        pltpu.async_copy(dst2_h.at[wid, ci], dv[b], isem[b])

    def wait_idx(cur, b):
        sv, dv, isem = sets[cur]
        pltpu.make_async_copy(src2_h.at[wid, 0], sv[b], isem[b]).wait()
        pltpu.make_async_copy(dst2_h.at[wid, 0], dv[b], isem[b]).wait()

    def issue_gathers(cur, b):
        sv, dv, _ = sets[cur]
        pltpu.async_copy(sd_h.at[sv[b]], sds[b], hsems[b])
        pltpu.async_copy(sd_h.at[dv[b]], sdd[b], hsems[b])
        pltpu.async_copy(tbl_h.at[sv[b]], rows[b], gsems[b])

    def compute_and_scatter(cur, b):
        sv, dv, _ = sets[cur]
        pltpu.make_async_copy(sd_h.at[sv[b]], sds[b], hsems[b]).wait()
        pltpu.make_async_copy(sd_h.at[dv[b]], sdd[b], hsems[b]).wait()
        offs = list(range(0, ka - 15, 16))
        if ka % 16:
            offs.append(ka - 16)
        for off in offs:
            ridx = iota16 + off
            sval = plsc.load_gather(sds[b], [ridx, zero16])
            dval = plsc.load_gather(sdd[b], [ridx, one16])
            e = sval + dval
            e = jnp.maximum(e, 0.2 * e)
            exb[b][pl.ds(off, 16)] = jnp.exp(e)
        pltpu.make_async_copy(tbl_h.at[sv[b]], rows[b], gsems[b]).wait()

        def scale(r, carry):
            cvec = plsc.load_gather(exb[b], [jnp.full((16,), r, jnp.int32)])
            for g in range(wa // 16):
                rows[b][r, pl.ds(g * 16, 16)] = (
                    rows[b][r, pl.ds(g * 16, 16)] * cvec)
            return carry

        lax.fori_loop(0, ka, scale, 0)
        pltpu.async_copy(rows[b], acc.at[dv[b]], ssems[b], add=True)

    def wait_scatter(b):
        pltpu.make_async_copy(rows[b], acc.at[dstv0[b]], ssems[b]).wait()

    def do_round(cur, g, first, issue_next):
        for b in range(NB):
            if not first:
                wait_scatter(b)
            if issue_next:
                issue_idx(1 - cur, b, (g + 1) * NB + b)
            wait_idx(cur, b)
            issue_gathers(cur, b)
        for b in range(NB):
            compute_and_scatter(cur, b)

    nround = EPT // ka // NB
    for b in range(NB):
        issue_idx(0, b, b)
    do_round(0, 0, True, True)

    npairs = (nround - 2) // 2

    def pair_body(i, carry):
        g = 2 * i + 1
        do_round(1, g, False, True)
        do_round(0, g + 1, False, True)
        return carry

    lax.fori_loop(0, npairs, pair_body, 0)
    for r in range(2 * npairs + 1, nround):
        do_round(r % 2, r, False, r < nround - 1)
    for b in range(NB):
        wait_scatter(b)

    plsc.subcore_barrier()
    pltpu.sync_copy(acc.at[pl.ds(s * RPT, RPT)], out_h.at[c, pl.ds(s * RPT, RPT)])


def _run_att(src2, dst2, sd, table, zeros_wa, width, ka):
    wa = width + 16
    f = pl.kernel(
        functools.partial(_sc_att, width, ka),
        out_type=jax.ShapeDtypeStruct((NC, NP, wa), jnp.float32),
        mesh=_mesh,
        compiler_params=_sc_params,
        scratch_types=[
            [pltpu.VMEM((ka,), jnp.int32) for _ in range(NB)],
            [pltpu.VMEM((ka,), jnp.int32) for _ in range(NB)],
            [pltpu.VMEM((ka,), jnp.int32) for _ in range(NB)],
            [pltpu.VMEM((ka,), jnp.int32) for _ in range(NB)],
            [pltpu.VMEM((ka, 16), jnp.float32) for _ in range(NB)],
            [pltpu.VMEM((ka, 16), jnp.float32) for _ in range(NB)],
            [pltpu.VMEM((ka,), jnp.float32) for _ in range(NB)],
            [pltpu.VMEM((ka, wa), jnp.float32) for _ in range(NB)],
            pltpu.VMEM_SHARED((NP, wa), jnp.float32),
            [pltpu.SemaphoreType.DMA for _ in range(NB)],
            [pltpu.SemaphoreType.DMA for _ in range(NB)],
            [pltpu.SemaphoreType.DMA for _ in range(NB)],
            [pltpu.SemaphoreType.DMA for _ in range(NB)],
            [pltpu.SemaphoreType.DMA for _ in range(NB)],
        ],
    )
    return f(src2, dst2, sd, table, zeros_wa)


BN = 1000
GRID = N // BN


def _row_spec(width):
    return pl.BlockSpec((BN, width), lambda i: (i, 0))


def _full_spec(shape):
    nd = len(shape)
    return pl.BlockSpec(shape, lambda i: (0,) * nd)


def _rsq_from(deg0_ref, deg1_ref):
    deg = jnp.maximum(deg0_ref[...] + deg1_ref[...], 1.0)
    return lax.rsqrt(deg)


def _ones_cols(base, width):
    bn = base.shape[0]
    return jnp.concatenate(
        [base, jnp.ones((bn, 1), jnp.float32),
         jnp.zeros((bn, 15), jnp.float32)], axis=1)


def _tc1(x_ref, wg1_ref, wx1_ref, bx1_ref, deg0_ref, deg1_ref,
         t1s_ref, hx1_ref):
    xb = x_ref[...]
    rsq = _rsq_from(deg0_ref, deg1_ref)
    t1s_ref[...] = jnp.dot(xb, wg1_ref[...],
                           preferred_element_type=jnp.float32) * rsq
    hx1 = jnp.dot(xb, wx1_ref[...],
                  preferred_element_type=jnp.float32) + bx1_ref[...]
    hx1_ref[...] = _ones_cols(hx1, 128)


def _tc2(p0_ref, p1_ref, bg1_ref, wg2_ref, deg0_ref, deg1_ref, t2s_ref):
    rsq = _rsq_from(deg0_ref, deg1_ref)
    h1 = jnp.maximum((p0_ref[...] + p1_ref[...]) * rsq + bg1_ref[...], 0.0)
    t2s_ref[...] = jnp.dot(h1, wg2_ref[...],
                           preferred_element_type=jnp.float32) * rsq


def _tc3(q0_ref, q1_ref, bg2_ref, wy1_ref, by1_ref, a1_ref,
         deg0_ref, deg1_ref, z_ref, hy1s_ref, sd1_ref):
    rsq = _rsq_from(deg0_ref, deg1_ref)
    z = (q0_ref[...] + q1_ref[...]) * rsq + bg2_ref[...]
    z_ref[...] = z
    zs = z - jnp.max(z, axis=-1, keepdims=True)
    ez = jnp.exp(zs)
    preds = ez / jnp.sum(ez, axis=-1, keepdims=True)
    hy1 = jnp.dot(preds, wy1_ref[...],
                  preferred_element_type=jnp.float32) + by1_ref[...]
    sd1_ref[...] = jnp.dot(hy1, a1_ref[...], preferred_element_type=jnp.float32)
    hy1s_ref[...] = hy1 * rsq


def _tc4(r0_ref, r1_ref, d10_ref, d11_ref, u0_ref, u1_ref,
         wx2_ref, bx2_ref, wy2_ref, by2_ref, a2_ref, deg0_ref, deg1_ref,
         hx2_ref, sd2_ref):
    rsq = _rsq_from(deg0_ref, deg1_ref)
    invd = 1.0 / (d10_ref[...] + d11_ref[...] + 1e-16)
    x1 = jnp.maximum((r0_ref[...] + r1_ref[...]) * invd, 0.0)
    hx2 = jnp.dot(x1, wx2_ref[...],
                  preferred_element_type=jnp.float32) + bx2_ref[...]
    hx2_ref[...] = _ones_cols(hx2, 64)
    y1 = jnp.maximum((u0_ref[...] + u1_ref[...]) * rsq, 0.0)
    hy2 = jnp.dot(y1, wy2_ref[...],
                  preferred_element_type=jnp.float32) + by2_ref[...]
    sd2_ref[...] = jnp.dot(hy2, a2_ref[...], preferred_element_type=jnp.float32)


def _tc5(z_ref, v0_ref, v1_ref, d20_ref, d21_ref, out_ref):
    invd = 1.0 / (d20_ref[...] + d21_ref[...] + 1e-16)
    out_ref[...] = (z_ref[...] + (v0_ref[...] + v1_ref[...]) * invd) * 0.5


def _sd_matrix(a_src, a_dst):
    return jnp.concatenate(
        [a_src.reshape(64, 1), a_dst.reshape(64, 1),
         jnp.zeros((64, 14), jnp.float32)], axis=1)


def kernel(x, edge_index, W_g1, b_g1, W_g2, b_g2, W_x1, b_x1, W_y1, b_y1,
           a_src1, a_dst1, W_x2, b_x2, W_y2, b_y2, a_src2, a_dst2):
    f32 = jnp.float32
    src = edge_index[0]
    dst = edge_index[1]
    srcD = src.reshape(NW, NCHD, KD)
    dstD = dst.reshape(NW, NCHD, KD)
    srcS1 = src.reshape(NW, EPT // KS128, KS128)
    dstS1 = dst.reshape(NW, EPT // KS128, KS128)
    srcS2 = src.reshape(NW, EPT // KS64, KS64)
    dstS2 = dst.reshape(NW, EPT // KS64, KS64)
    srcA1 = src.reshape(NW, EPT // KA128, KA128)
    dstA1 = dst.reshape(NW, EPT // KA128, KA128)
    srcA2 = src.reshape(NW, EPT // KA64, KA64)
    dstA2 = dst.reshape(NW, EPT // KA64, KA64)

    zeros64 = jnp.zeros((NP, 64), f32)
    zeros80 = jnp.zeros((NP, 80), f32)
    zeros128 = jnp.zeros((NP, 128), f32)
    zeros144 = jnp.zeros((NP, 144), f32)

    degp = _run_deg(dstD)
    deg0 = degp[0, :N].reshape(N, 1)
    deg1 = degp[1, :N].reshape(N, 1)

    t1s, hx1 = pl.pallas_call(
        _tc1,
        grid=(GRID,),
        in_specs=[
            _row_spec(128), _full_spec((128, 128)), _full_spec((128, 128)),
            _full_spec((1, 128)), _row_spec(1), _row_spec(1),
        ],
        out_specs=[_row_spec(128), _row_spec(144)],
        out_shape=[
            jax.ShapeDtypeStruct((N, 128), f32),
            jax.ShapeDtypeStruct((N, 144), f32),
        ],
    )(x, W_g1, W_x1, b_x1.reshape(1, 128), deg0, deg1)

    p = _run_spmm(srcS1, dstS1, t1s, zeros128, 128, KS128)

    t2s = pl.pallas_call(
        _tc2,
        grid=(GRID,),
        in_specs=[
            _row_spec(128), _row_spec(128), _full_spec((1, 128)),
            _full_spec((128, 64)), _row_spec(1), _row_spec(1),
        ],
        out_specs=_row_spec(64),
        out_shape=jax.ShapeDtypeStruct((N, 64), f32),
    )(p[0, :N], p[1, :N], b_g1.reshape(1, 128), W_g2, deg0, deg1)

    q = _run_spmm(srcS2, dstS2, t2s, zeros64, 64, KS64)

    a1 = _sd_matrix(a_src1, a_dst1)
    z, hy1s, sd1 = pl.pallas_call(
        _tc3,
        grid=(GRID,),
        in_specs=[
            _row_spec(64), _row_spec(64), _full_spec((1, 64)),
            _full_spec((64, 64)), _full_spec((1, 64)), _full_spec((64, 16)),
            _row_spec(1), _row_spec(1),
        ],
        out_specs=[_row_spec(64), _row_spec(64), _row_spec(16)],
        out_shape=[
            jax.ShapeDtypeStruct((N, 64), f32),
            jax.ShapeDtypeStruct((N, 64), f32),
            jax.ShapeDtypeStruct((N, 16), f32),
        ],
    )(q[0, :N], q[1, :N], b_g2.reshape(1, 64), W_y1, b_y1.reshape(1, 64), a1,
      deg0, deg1)

    r = _run_att(srcA1, dstA1, sd1, hx1, zeros144, 128, KA128)
    u = _run_spmm(srcS2, dstS2, hy1s, zeros64, 64, KS64)

    r0 = r[0, :N, :128]
    r1 = r[1, :N, :128]
    den10 = r[0, :N, 128].reshape(N, 1)
    den11 = r[1, :N, 128].reshape(N, 1)

    a2 = _sd_matrix(a_src2, a_dst2)
    hx2, sd2 = pl.pallas_call(
        _tc4,
        grid=(GRID,),
        in_specs=[
            _row_spec(128), _row_spec(128), _row_spec(1), _row_spec(1),
            _row_spec(64), _row_spec(64),
            _full_spec((128, 64)), _full_spec((1, 64)),
            _full_spec((64, 64)), _full_spec((1, 64)), _full_spec((64, 16)),
            _row_spec(1), _row_spec(1),
        ],
        out_specs=[_row_spec(80), _row_spec(16)],
        out_shape=[
            jax.ShapeDtypeStruct((N, 80), f32),
            jax.ShapeDtypeStruct((N, 16), f32),
        ],
    )(r0, r1, den10, den11, u[0, :N], u[1, :N],
      W_x2, b_x2.reshape(1, 64), W_y2, b_y2.reshape(1, 64), a2, deg0, deg1)

    v = _run_att(srcA2, dstA2, sd2, hx2, zeros80, 64, KA64)

    v0 = v[0, :N, :64]
    v1 = v[1, :N, :64]
    den20 = v[0, :N, 64].reshape(N, 1)
    den21 = v[1, :N, 64].reshape(N, 1)

    out = pl.pallas_call(
        _tc5,
        grid=(GRID,),
        in_specs=[
            _row_spec(64), _row_spec(64), _row_spec(64),
            _row_spec(1), _row_spec(1),
        ],
        out_specs=_row_spec(64),
        out_shape=jax.ShapeDtypeStruct((N, 64), f32),
    )(z, v0, v1, den20, den21)
    return out


# speedup vs baseline: 37.1790x; 1.0289x over previous
import functools

import jax
import jax.numpy as jnp
from jax import lax
from jax.experimental import pallas as pl
from jax.experimental.pallas import tpu as pltpu
from jax.experimental.pallas import tpu_sc as plsc

N = 10000
NP = 10240
E = 320000

NC = 2
NS = 16
NW = NC * NS
EPT = E // NW
RPT = NP // NS

KD = 80
NCHD = EPT // KD
KS128 = 40
KS64 = 80
DEPTH = 8
KA128 = 40
KA64 = 80
NB = 5

_mesh = plsc.VectorSubcoreMesh(
    core_axis_name="c", subcore_axis_name="s", num_cores=NC, num_subcores=NS)

_sc_params = pltpu.CompilerParams(
    use_tc_tiling_on_sc=False, needs_layout_passes=False)


def _zero_vmem_1d(ref, n):
    zv = jnp.zeros((16,), jnp.float32)

    def body(i, carry):
        ref[pl.ds(i * 16, 16)] = zv
        return carry

    lax.fori_loop(0, n // 16, body, 0)


def _sc_deg(dst2_h, out_h, dstv, deg_v, stage, tmp_v, out_v, sem):
    c = lax.axis_index("c")
    s = lax.axis_index("s")
    wid = c * NS + s
    pltpu.sync_copy(dst2_h.at[wid], dstv)
    _zero_vmem_1d(deg_v, NP)
    ones = jnp.ones((16,), jnp.float32)

    def step(i, carry):
        for j in range(KD // 16):
            di = dstv[i, pl.ds(j * 16, 16)]
            plsc.addupdate_scatter(deg_v, [di], ones)
        return carry

    lax.fori_loop(0, NCHD, step, 0)

    pltpu.sync_copy(deg_v, stage.at[s])
    plsc.subcore_barrier()
    pltpu.sync_copy(stage.at[:, pl.ds(s * RPT, RPT)], tmp_v)

    def red(w, carry):
        acc16 = tmp_v[0, pl.ds(w * 16, 16)]
        for t in range(1, NS):
            acc16 = acc16 + tmp_v[t, pl.ds(w * 16, 16)]
        out_v[pl.ds(w * 16, 16)] = acc16
        return carry

    lax.fori_loop(0, RPT // 16, red, 0)
    pltpu.sync_copy(out_v, out_h.at[c, pl.ds(s * RPT, RPT)])


def _run_deg(dst2):
    f = pl.kernel(
        _sc_deg,
        out_type=jax.ShapeDtypeStruct((NC, NP), jnp.float32),
        mesh=_mesh,
        compiler_params=_sc_params,
        scratch_types=[
            pltpu.VMEM((NCHD, KD), jnp.int32),
            pltpu.VMEM((NP,), jnp.float32),
            pltpu.VMEM_SHARED((NS, NP), jnp.float32),
            pltpu.VMEM((NS, RPT), jnp.float32),
            pltpu.VMEM((RPT,), jnp.float32),
            pltpu.SemaphoreType.DMA,
        ],
    )
    return f(dst2)


def _sc_spmm(width, ks, idx2_h, tbl_h, zeros_h, out_h,
             idxv, rows, acc, isems, gsems, ssems):
    c = lax.axis_index("c")
    s = lax.axis_index("s")
    wid = c * NS + s
    pltpu.sync_copy(zeros_h.at[pl.ds(s * RPT, RPT)], acc.at[pl.ds(s * RPT, RPT)])
    plsc.subcore_barrier()

    def issue_idx(ci):
        b = ci % DEPTH
        pltpu.async_copy(idx2_h.at[wid, ci], idxv[b], isems[b])

    def issue_gather(ci):
        b = ci % DEPTH
        pltpu.make_async_copy(idx2_h.at[wid, 0], idxv[b], isems[b]).wait()
        pltpu.async_copy(tbl_h.at[idxv[b].at[0]], rows[b], gsems[b])

    def issue_scatter(ci):
        b = ci % DEPTH
        pltpu.make_async_copy(tbl_h.at[idxv[b].at[0]], rows[b], gsems[b]).wait()
        pltpu.async_copy(rows[b], acc.at[idxv[b].at[1]], ssems[b], add=True)

    def wait_scatter(ci):
        b = ci % DEPTH
        pltpu.make_async_copy(rows[b], acc.at[idxv[b].at[1]], ssems[b]).wait()

    nchs = EPT // ks
    for t in range(nchs + 5):
        c0 = t
        if c0 < nchs:
            if c0 >= DEPTH:
                wait_scatter(c0 - DEPTH)
            issue_idx(c0)
        c1 = t - 2
        if 0 <= c1 < nchs:
            issue_gather(c1)
        c2 = t - 5
        if 0 <= c2 < nchs:
            issue_scatter(c2)
    for ci in range(nchs - DEPTH, nchs):
        wait_scatter(ci)
    plsc.subcore_barrier()
    pltpu.sync_copy(acc.at[pl.ds(s * RPT, RPT)], out_h.at[c, pl.ds(s * RPT, RPT)])


def _run_spmm(idx2, table, zeros_w, width, ks):
    f = pl.kernel(
        functools.partial(_sc_spmm, width, ks),
        out_type=jax.ShapeDtypeStruct((NC, NP, width), jnp.float32),
        mesh=_mesh,
        compiler_params=_sc_params,
        scratch_types=[
            [pltpu.VMEM((2, ks), jnp.int32) for _ in range(DEPTH)],
            [pltpu.VMEM((ks, width), jnp.float32) for _ in range(DEPTH)],
            pltpu.VMEM_SHARED((NP, width), jnp.float32),
            [pltpu.SemaphoreType.DMA for _ in range(DEPTH)],
            [pltpu.SemaphoreType.DMA for _ in range(DEPTH)],
            [pltpu.SemaphoreType.DMA for _ in range(DEPTH)],
        ],
    )
    return f(idx2, table, zeros_w)


def _sc_att(width, ka, idx2_h, sd_h, tbl_h, zeros_h, out_h,
            idxv0, idxv1, sdb, exb, rows, acc,
            isems0, isems1, hsems, gsems, ssems):
    wa = width + 16
    c = lax.axis_index("c")
    s = lax.axis_index("s")
    wid = c * NS + s
    pltpu.sync_copy(zeros_h.at[pl.ds(s * RPT, RPT)], acc.at[pl.ds(s * RPT, RPT)])
    plsc.subcore_barrier()

    iota16 = lax.iota(jnp.int32, 16)
    zero16 = jnp.zeros((16,), jnp.int32)
    one16 = jnp.ones((16,), jnp.int32)
    sets = ((idxv0, isems0), (idxv1, isems1))

    def issue_idx(cur, b, ci):
        iv, isem = sets[cur]
        pltpu.async_copy(idx2_h.at[wid, ci], iv[b], isem[b])

    def wait_idx(cur, b):
        iv, isem = sets[cur]
        pltpu.make_async_copy(idx2_h.at[wid, 0], iv[b], isem[b]).wait()

    def issue_gathers(cur, b):
        iv, _ = sets[cur]
        pltpu.async_copy(sd_h.at[iv[b].at[0]], sdb[b].at[0], hsems[b])
        pltpu.async_copy(sd_h.at[iv[b].at[1]], sdb[b].at[1], hsems[b])
        pltpu.async_copy(tbl_h.at[iv[b].at[0]], rows[b], gsems[b])

    def compute_and_scatter(cur, b):
        iv, _ = sets[cur]
        pltpu.make_async_copy(sd_h.at[iv[b].at[0]], sdb[b].at[0], hsems[b]).wait()
        pltpu.make_async_copy(sd_h.at[iv[b].at[1]], sdb[b].at[1], hsems[b]).wait()
        offs = list(range(0, ka - 15, 16))
        if ka % 16:
            offs.append(ka - 16)
        for off in offs:
            ridx = iota16 + off
            sval = plsc.load_gather(sdb[b], [zero16, ridx, zero16])
            dval = plsc.load_gather(sdb[b], [one16, ridx, one16])
            e = sval + dval
            e = jnp.maximum(e, 0.2 * e)
            exb[b][pl.ds(off, 16)] = jnp.exp(e)
        pltpu.make_async_copy(tbl_h.at[iv[b].at[0]], rows[b], gsems[b]).wait()

        def scale(r):
            cvec = plsc.load_gather(exb[b], [jnp.full((16,), r, jnp.int32)])
            for g in range(wa // 16):
                rows[b][r, pl.ds(g * 16, 16)] = (
                    rows[b][r, pl.ds(g * 16, 16)] * cvec)

        plsc.parallel_loop(0, ka, 1, unroll=2)(scale)

        pltpu.async_copy(rows[b], acc.at[iv[b].at[1]], ssems[b], add=True)

    def wait_scatter(b):
        pltpu.make_async_copy(rows[b], acc.at[idxv0[b].at[1]], ssems[b]).wait()

    def do_round(cur, g, first, issue_next):
        for b in range(NB):
            if not first:
                wait_scatter(b)
            if issue_next:
                issue_idx(1 - cur, b, (g + 1) * NB + b)
            wait_idx(cur, b)
            issue_gathers(cur, b)
        for b in range(NB):
            compute_and_scatter(cur, b)

    nround = EPT // ka // NB
    for b in range(NB):
        issue_idx(0, b, b)
    do_round(0, 0, True, True)

    npairs = (nround - 2) // 2

    def pair_body(i, carry):
        g = 2 * i + 1
        do_round(1, g, False, True)
        do_round(0, g + 1, False, True)
        return carry

    lax.fori_loop(0, npairs, pair_body, 0)
    for r in range(2 * npairs + 1, nround):
        do_round(r % 2, r, False, r < nround - 1)
    for b in range(NB):
        wait_scatter(b)

    plsc.subcore_barrier()
    pltpu.sync_copy(acc.at[pl.ds(s * RPT, RPT)], out_h.at[c, pl.ds(s * RPT, RPT)])


def _run_att(idx2, sd, table, zeros_wa, width, ka):
    wa = width + 16
    f = pl.kernel(
        functools.partial(_sc_att, width, ka),
        out_type=jax.ShapeDtypeStruct((NC, NP, wa), jnp.float32),
        mesh=_mesh,
        compiler_params=_sc_params,
        scratch_types=[
            [pltpu.VMEM((2, ka), jnp.int32) for _ in range(NB)],
            [pltpu.VMEM((2, ka), jnp.int32) for _ in range(NB)],
            [pltpu.VMEM((2, ka, 16), jnp.float32) for _ in range(NB)],
            [pltpu.VMEM((ka,), jnp.float32) for _ in range(NB)],
            [pltpu.VMEM((ka, wa), jnp.float32) for _ in range(NB)],
            pltpu.VMEM_SHARED((NP, wa), jnp.float32),
            [pltpu.SemaphoreType.DMA for _ in range(NB)],
            [pltpu.SemaphoreType.DMA for _ in range(NB)],
            [pltpu.SemaphoreType.DMA for _ in range(NB)],
            [pltpu.SemaphoreType.DMA for _ in range(NB)],
            [pltpu.SemaphoreType.DMA for _ in range(NB)],
        ],
    )
    return f(idx2, sd, table, zeros_wa)


BN = 1000
GRID = N // BN


def _row_spec(width):
    return pl.BlockSpec((BN, width), lambda i: (i, 0))


def _full_spec(shape):
    nd = len(shape)
    return pl.BlockSpec(shape, lambda i: (0,) * nd)


def _rsq_from(deg0_ref, deg1_ref):
    deg = jnp.maximum(deg0_ref[...] + deg1_ref[...], 1.0)
    return lax.rsqrt(deg)


def _ones_cols(base, width):
    bn = base.shape[0]
    return jnp.concatenate(
        [base, jnp.ones((bn, 1), jnp.float32),
         jnp.zeros((bn, 15), jnp.float32)], axis=1)


def _tc1(x_ref, wg1_ref, wx1_ref, bx1_ref, deg0_ref, deg1_ref,
         t1s_ref, hx1_ref):
    xb = x_ref[...]
    rsq = _rsq_from(deg0_ref, deg1_ref)
    t1s_ref[...] = jnp.dot(xb, wg1_ref[...],
                           preferred_element_type=jnp.float32) * rsq
    hx1 = jnp.dot(xb, wx1_ref[...],
                  preferred_element_type=jnp.float32) + bx1_ref[...]
    hx1_ref[...] = _ones_cols(hx1, 128)


def _tc2(p0_ref, p1_ref, bg1_ref, wg2_ref, deg0_ref, deg1_ref, t2s_ref):
    rsq = _rsq_from(deg0_ref, deg1_ref)
    h1 = jnp.maximum((p0_ref[...] + p1_ref[...]) * rsq + bg1_ref[...], 0.0)
    t2s_ref[...] = jnp.dot(h1, wg2_ref[...],
                           preferred_element_type=jnp.float32) * rsq


def _tc3(q0_ref, q1_ref, bg2_ref, wy1_ref, by1_ref, a1_ref,
         deg0_ref, deg1_ref, z_ref, hy1s_ref, sd1_ref):
    rsq = _rsq_from(deg0_ref, deg1_ref)
    z = (q0_ref[...] + q1_ref[...]) * rsq + bg2_ref[...]
    z_ref[...] = z
    zs = z - jnp.max(z, axis=-1, keepdims=True)
    ez = jnp.exp(zs)
    preds = ez / jnp.sum(ez, axis=-1, keepdims=True)
    hy1 = jnp.dot(preds, wy1_ref[...],
                  preferred_element_type=jnp.float32) + by1_ref[...]
    sd1_ref[...] = jnp.dot(hy1, a1_ref[...], preferred_element_type=jnp.float32)
    hy1s_ref[...] = hy1 * rsq


def _tc4(r0_ref, r1_ref, d10_ref, d11_ref, u0_ref, u1_ref,
         wx2_ref, bx2_ref, wy2_ref, by2_ref, a2_ref, deg0_ref, deg1_ref,
         hx2_ref, sd2_ref):
    rsq = _rsq_from(deg0_ref, deg1_ref)
    invd = 1.0 / (d10_ref[...] + d11_ref[...] + 1e-16)
    x1 = jnp.maximum((r0_ref[...] + r1_ref[...]) * invd, 0.0)
    hx2 = jnp.dot(x1, wx2_ref[...],
                  preferred_element_type=jnp.float32) + bx2_ref[...]
    hx2_ref[...] = _ones_cols(hx2, 64)
    y1 = jnp.maximum((u0_ref[...] + u1_ref[...]) * rsq, 0.0)
    hy2 = jnp.dot(y1, wy2_ref[...],
                  preferred_element_type=jnp.float32) + by2_ref[...]
    sd2_ref[...] = jnp.dot(hy2, a2_ref[...], preferred_element_type=jnp.float32)


def _tc5(z_ref, v0_ref, v1_ref, d20_ref, d21_ref, out_ref):
    invd = 1.0 / (d20_ref[...] + d21_ref[...] + 1e-16)
    out_ref[...] = (z_ref[...] + (v0_ref[...] + v1_ref[...]) * invd) * 0.5


def _sd_matrix(a_src, a_dst):
    return jnp.concatenate(
        [a_src.reshape(64, 1), a_dst.reshape(64, 1),
         jnp.zeros((64, 14), jnp.float32)], axis=1)


def kernel(x, edge_index, W_g1, b_g1, W_g2, b_g2, W_x1, b_x1, W_y1, b_y1,
           a_src1, a_dst1, W_x2, b_x2, W_y2, b_y2, a_src2, a_dst2):
    f32 = jnp.float32
    src = edge_index[0]
    dst = edge_index[1]
    srcD = src.reshape(NW, NCHD, KD)
    dstD = dst.reshape(NW, NCHD, KD)
    def _pair(k):
        return jnp.stack([src.reshape(NW, EPT // k, k),
                          dst.reshape(NW, EPT // k, k)], axis=2)

    idxS1 = _pair(KS128)
    idxS2 = _pair(KS64)
    idxA1 = _pair(KA128)
    idxA2 = _pair(KA64)

    zeros64 = jnp.zeros((NP, 64), f32)
    zeros80 = jnp.zeros((NP, 80), f32)
    zeros128 = jnp.zeros((NP, 128), f32)
    zeros144 = jnp.zeros((NP, 144), f32)

    degp = _run_deg(dstD)
    deg0 = degp[0, :N].reshape(N, 1)
    deg1 = degp[1, :N].reshape(N, 1)

    t1s, hx1 = pl.pallas_call(
        _tc1,
        grid=(GRID,),
        in_specs=[
            _row_spec(128), _full_spec((128, 128)), _full_spec((128, 128)),
            _full_spec((1, 128)), _row_spec(1), _row_spec(1),
        ],
        out_specs=[_row_spec(128), _row_spec(144)],
        out_shape=[
            jax.ShapeDtypeStruct((N, 128), f32),
            jax.ShapeDtypeStruct((N, 144), f32),
        ],
    )(x, W_g1, W_x1, b_x1.reshape(1, 128), deg0, deg1)

    p = _run_spmm(idxS1, t1s, zeros128, 128, KS128)

    t2s = pl.pallas_call(
        _tc2,
        grid=(GRID,),
        in_specs=[
            _row_spec(128), _row_spec(128), _full_spec((1, 128)),
            _full_spec((128, 64)), _row_spec(1), _row_spec(1),
        ],
        out_specs=_row_spec(64),
        out_shape=jax.ShapeDtypeStruct((N, 64), f32),
    )(p[0, :N], p[1, :N], b_g1.reshape(1, 128), W_g2, deg0, deg1)

    q = _run_spmm(idxS2, t2s, zeros64, 64, KS64)

    a1 = _sd_matrix(a_src1, a_dst1)
    z, hy1s, sd1 = pl.pallas_call(
        _tc3,
        grid=(GRID,),
        in_specs=[
            _row_spec(64), _row_spec(64), _full_spec((1, 64)),
            _full_spec((64, 64)), _full_spec((1, 64)), _full_spec((64, 16)),
            _row_spec(1), _row_spec(1),
        ],
        out_specs=[_row_spec(64), _row_spec(64), _row_spec(16)],
        out_shape=[
            jax.ShapeDtypeStruct((N, 64), f32),
            jax.ShapeDtypeStruct((N, 64), f32),
            jax.ShapeDtypeStruct((N, 16), f32),
        ],
    )(q[0, :N], q[1, :N], b_g2.reshape(1, 64), W_y1, b_y1.reshape(1, 64), a1,
      deg0, deg1)

    r = _run_att(idxA1, sd1, hx1, zeros144, 128, KA128)
    u = _run_spmm(idxS2, hy1s, zeros64, 64, KS64)

    r0 = r[0, :N, :128]
    r1 = r[1, :N, :128]
    den10 = r[0, :N, 128].reshape(N, 1)
    den11 = r[1, :N, 128].reshape(N, 1)

    a2 = _sd_matrix(a_src2, a_dst2)
    hx2, sd2 = pl.pallas_call(
        _tc4,
        grid=(GRID,),
        in_specs=[
            _row_spec(128), _row_spec(128), _row_spec(1), _row_spec(1),
            _row_spec(64), _row_spec(64),
            _full_spec((128, 64)), _full_spec((1, 64)),
            _full_spec((64, 64)), _full_spec((1, 64)), _full_spec((64, 16)),
            _row_spec(1), _row_spec(1),
        ],
        out_specs=[_row_spec(80), _row_spec(16)],
        out_shape=[
            jax.ShapeDtypeStruct((N, 80), f32),
            jax.ShapeDtypeStruct((N, 16), f32),
        ],
    )(r0, r1, den10, den11, u[0, :N], u[1, :N],
      W_x2, b_x2.reshape(1, 64), W_y2, b_y2.reshape(1, 64), a2, deg0, deg1)

    v = _run_att(idxA2, sd2, hx2, zeros80, 64, KA64)

    v0 = v[0, :N, :64]
    v1 = v[1, :N, :64]
    den20 = v[0, :N, 64].reshape(N, 1)
    den21 = v[1, :N, 64].reshape(N, 1)

    out = pl.pallas_call(
        _tc5,
        grid=(GRID,),
        in_specs=[
            _row_spec(64), _row_spec(64), _row_spec(64),
            _row_spec(1), _row_spec(1),
        ],
        out_specs=_row_spec(64),
        out_shape=jax.ShapeDtypeStruct((N, 64), f32),
    )(z, v0, v1, den20, den21)
    return out


# speedup vs baseline: 39.3282x; 1.0578x over previous
import functools

import jax
import jax.numpy as jnp
from jax import lax
from jax.experimental import pallas as pl
from jax.experimental.pallas import tpu as pltpu
from jax.experimental.pallas import tpu_sc as plsc

N = 10000
NP = 10240
E = 320000

NC = 2
NS = 16
NW = NC * NS
EPT = E // NW
RPT = NP // NS

KD = 80
NCHD = EPT // KD
KS64 = 80
DEPTH = 8
KA128 = 40
KA64 = 80
NB = 5

_mesh = plsc.VectorSubcoreMesh(
    core_axis_name="c", subcore_axis_name="s", num_cores=NC, num_subcores=NS)

_sc_params = pltpu.CompilerParams(
    use_tc_tiling_on_sc=False, needs_layout_passes=False)


def _zero_vmem_1d(ref, n):
    zv = jnp.zeros((16,), jnp.float32)

    def body(i, carry):
        ref[pl.ds(i * 16, 16)] = zv
        return carry

    lax.fori_loop(0, n // 16, body, 0)


def _sc_deg(dst2_h, out_h, dstv, deg_v, stage, tmp_v, out_v, sem):
    c = lax.axis_index("c")
    s = lax.axis_index("s")
    wid = c * NS + s
    pltpu.sync_copy(dst2_h.at[wid], dstv)
    _zero_vmem_1d(deg_v, NP)
    ones = jnp.ones((16,), jnp.float32)

    def step(i, carry):
        for j in range(KD // 16):
            di = dstv[i, pl.ds(j * 16, 16)]
            plsc.addupdate_scatter(deg_v, [di], ones)
        return carry

    lax.fori_loop(0, NCHD, step, 0)

    pltpu.sync_copy(deg_v, stage.at[s])
    plsc.subcore_barrier()
    pltpu.sync_copy(stage.at[:, pl.ds(s * RPT, RPT)], tmp_v)

    def red(w, carry):
        acc16 = tmp_v[0, pl.ds(w * 16, 16)]
        for t in range(1, NS):
            acc16 = acc16 + tmp_v[t, pl.ds(w * 16, 16)]
        out_v[pl.ds(w * 16, 16)] = acc16
        return carry

    lax.fori_loop(0, RPT // 16, red, 0)
    pltpu.sync_copy(out_v, out_h.at[c, pl.ds(s * RPT, RPT)])


def _run_deg(dst2):
    f = pl.kernel(
        _sc_deg,
        out_type=jax.ShapeDtypeStruct((NC, NP), jnp.float32),
        mesh=_mesh,
        compiler_params=_sc_params,
        scratch_types=[
            pltpu.VMEM((NCHD, KD), jnp.int32),
            pltpu.VMEM((NP,), jnp.float32),
            pltpu.VMEM_SHARED((NS, NP), jnp.float32),
            pltpu.VMEM((NS, RPT), jnp.float32),
            pltpu.VMEM((RPT,), jnp.float32),
            pltpu.SemaphoreType.DMA,
        ],
    )
    return f(dst2)


def _sc_spmm(width, ks, depth, lag_g, lag_s, idx2_h, tbl_h, zeros_h, out_h,
             idxv, rows, acc, isems, gsems, ssems):
    c = lax.axis_index("c")
    s = lax.axis_index("s")
    wid = c * NS + s
    pltpu.sync_copy(zeros_h.at[pl.ds(s * RPT, RPT), pl.ds(0, width)],
                    acc.at[pl.ds(s * RPT, RPT)])
    plsc.subcore_barrier()

    def issue_idx(ci):
        b = ci % depth
        pltpu.async_copy(idx2_h.at[wid, ci], idxv[b], isems[b])

    def issue_gather(ci):
        b = ci % depth
        pltpu.make_async_copy(idx2_h.at[wid, 0], idxv[b], isems[b]).wait()
        pltpu.async_copy(tbl_h.at[idxv[b].at[0]], rows[b], gsems[b])

    def issue_scatter(ci):
        b = ci % depth
        pltpu.make_async_copy(tbl_h.at[idxv[b].at[0]], rows[b], gsems[b]).wait()
        pltpu.async_copy(rows[b], acc.at[idxv[b].at[1]], ssems[b], add=True)

    def wait_scatter(ci):
        b = ci % depth
        pltpu.make_async_copy(rows[b], acc.at[idxv[b].at[1]], ssems[b]).wait()

    nchs = EPT // ks
    for t in range(nchs + lag_s):
        c0 = t
        if c0 < nchs:
            if c0 >= depth:
                wait_scatter(c0 - depth)
            issue_idx(c0)
        c1 = t - lag_g
        if 0 <= c1 < nchs:
            issue_gather(c1)
        c2 = t - lag_s
        if 0 <= c2 < nchs:
            issue_scatter(c2)
    for ci in range(nchs - depth, nchs):
        wait_scatter(ci)
    plsc.subcore_barrier()
    pltpu.sync_copy(acc.at[pl.ds(s * RPT, RPT)], out_h.at[c, pl.ds(s * RPT, RPT)])


def _run_spmm(idx2, table, zeros_w, width, ks, depth=DEPTH, lag_g=2, lag_s=5):
    f = pl.kernel(
        functools.partial(_sc_spmm, width, ks, depth, lag_g, lag_s),
        out_type=jax.ShapeDtypeStruct((NC, NP, width), jnp.float32),
        mesh=_mesh,
        compiler_params=_sc_params,
        scratch_types=[
            [pltpu.VMEM((2, ks), jnp.int32) for _ in range(depth)],
            [pltpu.VMEM((ks, width), jnp.float32) for _ in range(depth)],
            pltpu.VMEM_SHARED((NP, width), jnp.float32),
            [pltpu.SemaphoreType.DMA for _ in range(depth)],
            [pltpu.SemaphoreType.DMA for _ in range(depth)],
            [pltpu.SemaphoreType.DMA for _ in range(depth)],
        ],
    )
    return f(idx2, table, zeros_w)


def _sc_att(width, ka, idx2_h, sd_h, tbl_h, zeros_h, out_h,
            idxv0, idxv1, sdb, exb, rows, acc,
            isems0, isems1, hsems, gsems, ssems):
    wa = width + 16
    c = lax.axis_index("c")
    s = lax.axis_index("s")
    wid = c * NS + s
    pltpu.sync_copy(zeros_h.at[pl.ds(s * RPT, RPT), pl.ds(0, wa)],
                    acc.at[pl.ds(s * RPT, RPT)])
    plsc.subcore_barrier()

    iota16 = lax.iota(jnp.int32, 16)
    zero16 = jnp.zeros((16,), jnp.int32)
    one16 = jnp.ones((16,), jnp.int32)
    sets = ((idxv0, isems0), (idxv1, isems1))

    def issue_idx(cur, b, ci):
        iv, isem = sets[cur]
        pltpu.async_copy(idx2_h.at[wid, ci], iv[b], isem[b])

    def wait_idx(cur, b):
        iv, isem = sets[cur]
        pltpu.make_async_copy(idx2_h.at[wid, 0], iv[b], isem[b]).wait()

    def issue_gathers(cur, b):
        iv, _ = sets[cur]
        pltpu.async_copy(sd_h.at[iv[b].at[1]], sdb[b], hsems[b])
        pltpu.async_copy(tbl_h.at[iv[b].at[0]], rows[b], gsems[b])

    def compute_and_scatter(cur, b):
        iv, _ = sets[cur]
        pltpu.make_async_copy(sd_h.at[iv[b].at[1]], sdb[b], hsems[b]).wait()
        pltpu.make_async_copy(tbl_h.at[iv[b].at[0]], rows[b], gsems[b]).wait()
        scol = jnp.full((16,), width + 1, jnp.int32)
        offs = list(range(0, ka - 15, 16))
        if ka % 16:
            offs.append(ka - 16)
        for off in offs:
            ridx = iota16 + off
            sval = plsc.load_gather(rows[b], [ridx, scol])
            dval = plsc.load_gather(sdb[b], [ridx, one16])
            e = sval + dval
            e = jnp.maximum(e, 0.2 * e)
            exb[b][pl.ds(off, 16)] = jnp.exp(e)

        def scale(r):
            cvec = plsc.load_gather(exb[b], [jnp.full((16,), r, jnp.int32)])
            for g in range(wa // 16):
                rows[b][r, pl.ds(g * 16, 16)] = (
                    rows[b][r, pl.ds(g * 16, 16)] * cvec)

        plsc.parallel_loop(0, ka, 1, unroll=4)(scale)

        pltpu.async_copy(rows[b], acc.at[iv[b].at[1]], ssems[b], add=True)

    def wait_scatter(b):
        pltpu.make_async_copy(rows[b], acc.at[idxv0[b].at[1]], ssems[b]).wait()

    def do_round(cur, g, first, issue_next):
        for b in range(NB):
            if not first:
                wait_scatter(b)
            if issue_next:
                issue_idx(1 - cur, b, (g + 1) * NB + b)
            wait_idx(cur, b)
            issue_gathers(cur, b)
        for b in range(NB):
            compute_and_scatter(cur, b)

    nround = EPT // ka // NB
    for b in range(NB):
        issue_idx(0, b, b)
    do_round(0, 0, True, True)

    npairs = (nround - 2) // 2

    def pair_body(i, carry):
        g = 2 * i + 1
        do_round(1, g, False, True)
        do_round(0, g + 1, False, True)
        return carry

    lax.fori_loop(0, npairs, pair_body, 0)
    for r in range(2 * npairs + 1, nround):
        do_round(r % 2, r, False, r < nround - 1)
    for b in range(NB):
        wait_scatter(b)

    plsc.subcore_barrier()
    pltpu.sync_copy(acc.at[pl.ds(s * RPT, RPT)], out_h.at[c, pl.ds(s * RPT, RPT)])


def _run_att(idx2, sd, table, zeros_wa, width, ka):
    wa = width + 16
    f = pl.kernel(
        functools.partial(_sc_att, width, ka),
        out_type=jax.ShapeDtypeStruct((NC, NP, wa), jnp.float32),
        mesh=_mesh,
        compiler_params=_sc_params,
        scratch_types=[
            [pltpu.VMEM((2, ka), jnp.int32) for _ in range(NB)],
            [pltpu.VMEM((2, ka), jnp.int32) for _ in range(NB)],
            [pltpu.VMEM((ka, 16), jnp.float32) for _ in range(NB)],
            [pltpu.VMEM((ka,), jnp.float32) for _ in range(NB)],
            [pltpu.VMEM((ka, wa), jnp.float32) for _ in range(NB)],
            pltpu.VMEM_SHARED((NP, wa), jnp.float32),
            [pltpu.SemaphoreType.DMA for _ in range(NB)],
            [pltpu.SemaphoreType.DMA for _ in range(NB)],
            [pltpu.SemaphoreType.DMA for _ in range(NB)],
            [pltpu.SemaphoreType.DMA for _ in range(NB)],
            [pltpu.SemaphoreType.DMA for _ in range(NB)],
        ],
    )
    return f(idx2, sd, table, zeros_wa)


BN = 1000
GRID = N // BN


def _row_spec(width):
    return pl.BlockSpec((BN, width), lambda i: (i, 0))


def _part_spec(width, half):
    return pl.BlockSpec((1, BN, width), lambda i, h=half: (h, i, 0))


def _full_spec(shape):
    nd = len(shape)
    return pl.BlockSpec(shape, lambda i: (0,) * nd)


def _rsq_from(deg0_ref, deg1_ref):
    deg = jnp.maximum(deg0_ref[...] + deg1_ref[...], 1.0)
    return lax.rsqrt(deg)


def _ones_cols(base, s_col):
    bn = base.shape[0]
    return jnp.concatenate(
        [base, jnp.ones((bn, 1), jnp.float32), s_col,
         jnp.zeros((bn, 14), jnp.float32)], axis=1)


def _tc1(x_ref, wg1_ref, deg0_ref, deg1_ref, t1s_ref):
    rsq = _rsq_from(deg0_ref, deg1_ref)
    t1s_ref[...] = jnp.dot(x_ref[...], wg1_ref[...],
                           preferred_element_type=jnp.float32) * rsq


def _tc2(p0_ref, p1_ref, bg1_ref, wg2_ref, deg0_ref, deg1_ref, t2s_ref):
    rsq = _rsq_from(deg0_ref, deg1_ref)
    h1 = jnp.maximum((p0_ref[0] + p1_ref[0]) * rsq + bg1_ref[...], 0.0)
    t2s_ref[...] = jnp.dot(h1, wg2_ref[...],
                           preferred_element_type=jnp.float32) * rsq


def _tc3(q0_ref, q1_ref, bg2_ref, wy1_ref, by1_ref, a1_ref,
         x_ref, wx1_ref, bx1_ref, deg0_ref, deg1_ref,
         z_ref, hy1s_ref, sd1_ref, hx1_ref):
    rsq = _rsq_from(deg0_ref, deg1_ref)
    z = (q0_ref[0] + q1_ref[0]) * rsq + bg2_ref[...]
    z_ref[...] = z
    zs = z - jnp.max(z, axis=-1, keepdims=True)
    ez = jnp.exp(zs)
    preds = ez / jnp.sum(ez, axis=-1, keepdims=True)
    hy1 = jnp.dot(preds, wy1_ref[...],
                  preferred_element_type=jnp.float32) + by1_ref[...]
    sd1 = jnp.dot(hy1, a1_ref[...], preferred_element_type=jnp.float32)
    sd1_ref[...] = sd1
    hy1s_ref[...] = hy1 * rsq
    hx1 = jnp.dot(x_ref[...], wx1_ref[...],
                  preferred_element_type=jnp.float32) + bx1_ref[...]
    hx1_ref[...] = _ones_cols(hx1, sd1[:, 0:1])


def _tc4(r0_ref, r1_ref, u0_ref, u1_ref,
         wx2_ref, bx2_ref, wy2_ref, by2_ref, a2_ref, deg0_ref, deg1_ref,
         hx2_ref, sd2_ref):
    rsq = _rsq_from(deg0_ref, deg1_ref)
    rsum = r0_ref[0] + r1_ref[0]
    invd = 1.0 / (rsum[:, 128:129] + 1e-16)
    x1 = jnp.maximum(rsum[:, :128] * invd, 0.0)
    hx2 = jnp.dot(x1, wx2_ref[...],
                  preferred_element_type=jnp.float32) + bx2_ref[...]
    y1 = jnp.maximum((u0_ref[0] + u1_ref[0]) * rsq, 0.0)
    hy2 = jnp.dot(y1, wy2_ref[...],
                  preferred_element_type=jnp.float32) + by2_ref[...]
    sd2 = jnp.dot(hy2, a2_ref[...], preferred_element_type=jnp.float32)
    sd2_ref[...] = sd2
    hx2_ref[...] = _ones_cols(hx2, sd2[:, 0:1])


def _tc5(z_ref, v0_ref, v1_ref, out_ref):
    vsum = v0_ref[0] + v1_ref[0]
    invd = 1.0 / (vsum[:, 64:65] + 1e-16)
    out_ref[...] = (z_ref[...] + vsum[:, :64] * invd) * 0.5


def _sd_matrix(a_src, a_dst):
    return jnp.concatenate(
        [a_src.reshape(64, 1), a_dst.reshape(64, 1),
         jnp.zeros((64, 14), jnp.float32)], axis=1)


def kernel(x, edge_index, W_g1, b_g1, W_g2, b_g2, W_x1, b_x1, W_y1, b_y1,
           a_src1, a_dst1, W_x2, b_x2, W_y2, b_y2, a_src2, a_dst2):
    f32 = jnp.float32
    src = edge_index[0]
    dst = edge_index[1]
    dstD = dst.reshape(NW, NCHD, KD)
    def _pair(k):
        return jnp.stack([src.reshape(NW, EPT // k, k),
                          dst.reshape(NW, EPT // k, k)], axis=2)

    idx40 = _pair(40)
    idx80 = _pair(80)

    zeros144 = jnp.zeros((NP, 144), f32)

    degp = _run_deg(dstD)
    deg0 = degp[0, :N].reshape(N, 1)
    deg1 = degp[1, :N].reshape(N, 1)

    t1s = pl.pallas_call(
        _tc1,
        grid=(GRID,),
        in_specs=[
            _row_spec(128), _full_spec((128, 128)), _row_spec(1), _row_spec(1),
        ],
        out_specs=_row_spec(128),
        out_shape=jax.ShapeDtypeStruct((N, 128), f32),
    )(x, W_g1, deg0, deg1)

    p = _run_spmm(idx80, t1s, zeros144, 128, 80, depth=4, lag_g=1, lag_s=3)

    t2s = pl.pallas_call(
        _tc2,
        grid=(GRID,),
        in_specs=[
            _part_spec(128, 0), _part_spec(128, 1), _full_spec((1, 128)),
            _full_spec((128, 64)), _row_spec(1), _row_spec(1),
        ],
        out_specs=_row_spec(64),
        out_shape=jax.ShapeDtypeStruct((N, 64), f32),
    )(p, p, b_g1.reshape(1, 128), W_g2, deg0, deg1)

    q = _run_spmm(idx80, t2s, zeros144, 64, KS64)

    a1 = _sd_matrix(a_src1, a_dst1)
    z, hy1s, sd1, hx1 = pl.pallas_call(
        _tc3,
        grid=(GRID,),
        in_specs=[
            _part_spec(64, 0), _part_spec(64, 1), _full_spec((1, 64)),
            _full_spec((64, 64)), _full_spec((1, 64)), _full_spec((64, 16)),
            _row_spec(128), _full_spec((128, 128)), _full_spec((1, 128)),
            _row_spec(1), _row_spec(1),
        ],
        out_specs=[_row_spec(64), _row_spec(64), _row_spec(16), _row_spec(144)],
        out_shape=[
            jax.ShapeDtypeStruct((N, 64), f32),
            jax.ShapeDtypeStruct((N, 64), f32),
            jax.ShapeDtypeStruct((N, 16), f32),
            jax.ShapeDtypeStruct((N, 144), f32),
        ],
    )(q, q, b_g2.reshape(1, 64), W_y1, b_y1.reshape(1, 64), a1,
      x, W_x1, b_x1.reshape(1, 128),
      deg0, deg1)

    r = _run_att(idx40, sd1, hx1, zeros144, 128, KA128)
    u = _run_spmm(idx80, hy1s, zeros144, 64, KS64)

    a2 = _sd_matrix(a_src2, a_dst2)
    hx2, sd2 = pl.pallas_call(
        _tc4,
        grid=(GRID,),
        in_specs=[
            _part_spec(144, 0), _part_spec(144, 1),
            _part_spec(64, 0), _part_spec(64, 1),
            _full_spec((128, 64)), _full_spec((1, 64)),
            _full_spec((64, 64)), _full_spec((1, 64)), _full_spec((64, 16)),
            _row_spec(1), _row_spec(1),
        ],
        out_specs=[_row_spec(80), _row_spec(16)],
        out_shape=[
            jax.ShapeDtypeStruct((N, 80), f32),
            jax.ShapeDtypeStruct((N, 16), f32),
        ],
    )(r, r, u, u,
      W_x2, b_x2.reshape(1, 64), W_y2, b_y2.reshape(1, 64), a2, deg0, deg1)

    v = _run_att(idx80, sd2, hx2, zeros144, 64, KA64)

    out = pl.pallas_call(
        _tc5,
        grid=(GRID,),
        in_specs=[
            _row_spec(64), _part_spec(80, 0), _part_spec(80, 1),
        ],
        out_specs=_row_spec(64),
        out_shape=jax.ShapeDtypeStruct((N, 64), f32),
    )(z, v, v)
    return out
